# Initial kernel scaffold; baseline (speedup 1.0000x reference)
import jax
import jax.numpy as jnp
from jax.experimental import pallas as pl


def kernel(feats, edge_index, W1, b1, W2, b2, Wp1, bp1, Wp2, bp2, Wp3, bp3, Wq1, bq1, gamma, beta, Wq2, bq2):
    raise NotImplementedError("write your pallas kernel here")


# speedup vs baseline: 24.3593x; 24.3593x over previous
import dataclasses
import functools

import jax
import jax.numpy as jnp
from jax import lax
from jax.experimental import pallas as pl
from jax.experimental.pallas import tpu as pltpu
from jax.experimental.pallas import tpu_sc as plsc

N = 100000
E = 3200000
NP = 100352
NW = 32
EPW = E // NW
CHK = 2000
NCHK = EPW // CHK
SBC = 4096
STG = SBC + CHK
CAP = ((EPW // SBC) + 1) * SBC
F = 18
FP = 32
SLW = 2048
WAVE = NW * SLW
NWAVES = 2

_f32 = jnp.float32
_i32 = jnp.int32


def _iota16():
  return lax.iota(_i32, 16)


def _full16(x, dtype=_i32):
  return jnp.full((16,), x, dtype)


def _rsqrt_raw(x):
  i = plsc.bitcast(x, _i32)
  i = jnp.int32(0x5F3759DF) - lax.shift_right_arithmetic(i, 1)
  y = plsc.bitcast(i, _f32)
  for _ in range(3):
    y = y * (1.5 - 0.5 * x * y * y)
  return y


def _rsqrt_deg(x):
  return _rsqrt_raw(jnp.maximum(x, 1.0))


@functools.lru_cache(maxsize=1)
def _build():
  _mesh = plsc.VectorSubcoreMesh(core_axis_name="c", subcore_axis_name="s")
  _cp = pltpu.CompilerParams()
  if "needs_layout_passes" in pltpu.CompilerParams.__dataclass_fields__:
    _cp = dataclasses.replace(_cp, needs_layout_passes=False)

  @functools.partial(
      pl.kernel,
      out_type=jax.ShapeDtypeStruct((NW * NP,), _f32),
      mesh=_mesh,
      compiler_params=_cp,
      scratch_types=[
          pltpu.VMEM((NP,), _f32),
          pltpu.VMEM((CHK,), _i32),
      ],
  )
  def _k_hist(esrc, hpart, hist, ibuf):
    cid = lax.axis_index("c")
    sid = lax.axis_index("s")
    w = cid * 16 + sid

    @pl.loop(0, NP, step=16)
    def _(i):
      hist[pl.ds(i, 16)] = jnp.zeros((16,), _f32)

    @pl.loop(0, NCHK)
    def _(c):
      pltpu.sync_copy(esrc.at[pl.ds(w * EPW + c * CHK, CHK)], ibuf)

      @pl.loop(0, CHK, step=16)
      def _(j):
        v = ibuf[pl.ds(j, 16)]
        cnt, last = plsc.scan_count(v)
        plsc.addupdate_scatter(hist, [v], cnt.astype(_f32), mask=last)

    pltpu.sync_copy(hist, hpart.at[pl.ds(w * NP, NP)])

  @functools.partial(
      pl.kernel,
      out_type=jax.ShapeDtypeStruct((NP,), _f32),
      mesh=_mesh,
      compiler_params=_cp,
      scratch_types=[
          pltpu.VMEM((32, 1024), _f32),
          pltpu.VMEM((1024,), _f32),
      ],
  )
  def _k_hred(hpart, dego, rbuf, abuf):
    cid = lax.axis_index("c")
    sid = lax.axis_index("s")

    def emit(base, chunks):
      off = 0
      for ln in chunks:
        loff = off

        @pl.loop(0, NW)
        def _(p, loff=loff, ln=ln):
          pltpu.sync_copy(hpart.at[pl.ds(p * NP + base + loff, ln)],
                          rbuf.at[p, pl.ds(0, ln)])

        @pl.loop(0, ln, step=16)
        def _(j):
          acc = lax.fori_loop(
              0, NW, lambda i, acc: acc + rbuf[i, pl.ds(j, 16)],
              jnp.zeros((16,), _f32))
          abuf[pl.ds(j, 16)] = acc

        pltpu.sync_copy(abuf.at[pl.ds(0, ln)], dego.at[pl.ds(base + off, ln)])
        off += ln

    @pl.when(cid == 0)
    def _():
      emit(sid * 3200, [1024, 1024, 1024, 128])

    @pl.when(cid == 1)
    def _():
      emit(16 * 3200 + sid * 3072, [1024, 1024, 1024])

  @functools.partial(
      pl.kernel,
      out_type=(jax.ShapeDtypeStruct((NW * CAP,), _i32),
                jax.ShapeDtypeStruct((NW * CAP,), _i32),
                jax.ShapeDtypeStruct((NW * 16,), _i32)),
      mesh=_mesh,
      compiler_params=_cp,
      scratch_types=[
          pltpu.VMEM((CHK,), _i32),
          pltpu.VMEM((CHK,), _i32),
          pltpu.VMEM((STG,), _i32),
          pltpu.VMEM((STG,), _i32),
          pltpu.VMEM((16,), _i32),
      ],
  )
  def _k_lists(esrc, edst, l0, l1, cnts, ibs, ibd, st0, st1, cbuf):
    cid = lax.axis_index("c")
    sid = lax.axis_index("s")
    w = cid * 16 + sid

    def chunk(c, carry):
      o0, f0, o1, f1 = carry
      base = w * EPW + c * CHK
      pltpu.sync_copy(esrc.at[pl.ds(base, CHK)], ibs)
      pltpu.sync_copy(edst.at[pl.ds(base, CHK)], ibd)

      def vreg(j, carry):
        o0, o1 = carry
        vd = ibd[pl.ds(j * 16, 16)]
        vs = ibs[pl.ds(j * 16, 16)]
        m0 = vd == 0
        m1 = vd == 1
        plsc.store_compressed(st0.at[pl.ds(o0, 16)], vs, mask=m0)
        plsc.store_compressed(st1.at[pl.ds(o1, 16)], vs, mask=m1)
        o0 = o0 + plsc.all_reduce_population_count(m0)[0]
        o1 = o1 + plsc.all_reduce_population_count(m1)[0]
        return o0, o1

      o0, o1 = lax.fori_loop(0, CHK // 16, vreg, (o0, o1))

      def flush0(o, f):
        pltpu.sync_copy(st0.at[pl.ds(0, SBC)],
                        l0.at[pl.ds(w * CAP + f * SBC, SBC)])

        @pl.loop(0, CHK, step=16)
        def _(j):
          st0[pl.ds(j, 16)] = st0[pl.ds(SBC + j, 16)]
        return o - SBC, f + 1

      def flush1(o, f):
        pltpu.sync_copy(st1.at[pl.ds(0, SBC)],
                        l1.at[pl.ds(w * CAP + f * SBC, SBC)])

        @pl.loop(0, CHK, step=16)
        def _(j):
          st1[pl.ds(j, 16)] = st1[pl.ds(SBC + j, 16)]
        return o - SBC, f + 1

      def keep(o, f):
        return o, f

      o0, f0 = lax.cond(o0 >= SBC, flush0, keep, o0, f0)
      o1, f1 = lax.cond(o1 >= SBC, flush1, keep, o1, f1)
      return o0, f0, o1, f1

    z = jnp.int32(0)
    o0, f0, o1, f1 = lax.fori_loop(0, NCHK, chunk, (z, z, z, z))
    pltpu.sync_copy(st0.at[pl.ds(0, SBC)],
                    l0.at[pl.ds(w * CAP + f0 * SBC, SBC)])
    pltpu.sync_copy(st1.at[pl.ds(0, SBC)],
                    l1.at[pl.ds(w * CAP + f1 * SBC, SBC)])
    tot0 = f0 * SBC + o0
    tot1 = f1 * SBC + o1
    it = _iota16()
    cbuf[pl.ds(0, 16)] = jnp.where(it == 0, tot0,
                                   jnp.where(it == 1, tot1, 0))
    pltpu.sync_copy(cbuf, cnts.at[pl.ds(w * 16, 16)])

  @functools.partial(
      pl.kernel,
      out_type=(jax.ShapeDtypeStruct((NW * CAP,), _i32),
                jax.ShapeDtypeStruct((NW * CAP,), _i32),
                jax.ShapeDtypeStruct((NW * 16,), _i32),
                jax.ShapeDtypeStruct((NW * CAP,), _i32),
                jax.ShapeDtypeStruct((NW * CAP,), _i32),
                jax.ShapeDtypeStruct((16,), _i32)),
      mesh=_mesh,
      compiler_params=_cp,
      scratch_types=[
          pltpu.VMEM((NP,), _i32),
          pltpu.VMEM((CHK,), _i32),
          pltpu.VMEM((CHK,), _i32),
          pltpu.VMEM((STG,), _i32),
          pltpu.VMEM((STG,), _i32),
          pltpu.VMEM((128,), _i32),
          pltpu.VMEM((128,), _i32),
          pltpu.VMEM((16,), _i32),
      ],
  )
  def _k_mslot(esrc, edst, l0, l1, cnts,
               msrc, mslt, mcnt, slt0, slt1, nsl,
               markv, ibs, ibd, sts, stq, lbuf, sbuf, cbuf):
    cid = lax.axis_index("c")
    sid = lax.axis_index("s")
    w = cid * 16 + sid
    it = _iota16()
    lane0 = it == 0

    @pl.loop(0, NP, step=16)
    def _(i):
      markv[pl.ds(i, 16)] = jnp.zeros((16,), _i32)

    nxt = jnp.int32(1)
    for lst, slout, lane in ((l0, slt0, 0), (l1, slt1, 1)):
      def region(r, nxt, lst=lst, slout=slout, lane=lane):
        pltpu.sync_copy(cnts.at[pl.ds(r * 16, 16)], cbuf)
        tot = cbuf[pl.ds(0, 16)][lane]
        nb = lax.shift_right_arithmetic(tot + 127, 7)

        def block(b, nxt, lst=lst, slout=slout, tot=tot):
          pltpu.sync_copy(lst.at[pl.ds(r * CAP + b * 128, 128)], lbuf)
          rem = jnp.minimum(tot - b * 128, 128)

          def entry(e, nxt):
            sspl = plsc.load_gather(lbuf, [_full16(e)])
            old = plsc.load_gather(markv, [sspl])
            isnew = old == 0
            slotspl = jnp.where(isnew, _full16(nxt), old)
            plsc.store_scatter(markv, [sspl], slotspl,
                               mask=jnp.logical_and(isnew, lane0))

            @pl.when(w == 0)
            def _():
              plsc.store_scatter(sbuf, [_full16(e)], slotspl - 1, mask=lane0)
            return nxt + isnew.astype(_i32)[0]

          nxt = lax.fori_loop(0, rem, entry, nxt)

          @pl.when(w == 0)
          def _():
            pltpu.sync_copy(sbuf, slout.at[pl.ds(r * CAP + b * 128, 128)])
          return nxt

        return lax.fori_loop(0, nb, block, nxt)

      nxt = lax.fori_loop(0, NW, region, nxt)

    @pl.when(w == 0)
    def _():
      cbuf[pl.ds(0, 16)] = jnp.where(lane0, nxt - 1, 0)
      pltpu.sync_copy(cbuf, nsl)

    def chunk(c, carry):
      o, f = carry
      base = w * EPW + c * CHK
      pltpu.sync_copy(esrc.at[pl.ds(base, CHK)], ibs)
      pltpu.sync_copy(edst.at[pl.ds(base, CHK)], ibd)

      def vreg(j, o):
        vd = ibd[pl.ds(j * 16, 16)]
        vs = ibs[pl.ds(j * 16, 16)]
        g = plsc.load_gather(markv, [vd])
        m = g > 0
        plsc.store_compressed(sts.at[pl.ds(o, 16)], vs, mask=m)
        plsc.store_compressed(stq.at[pl.ds(o, 16)], g - 1, mask=m)
        return o + plsc.all_reduce_population_count(m)[0]

      o = lax.fori_loop(0, CHK // 16, vreg, o)

      def flush(o, f):
        pltpu.sync_copy(sts.at[pl.ds(0, SBC)],
                        msrc.at[pl.ds(w * CAP + f * SBC, SBC)])
        pltpu.sync_copy(stq.at[pl.ds(0, SBC)],
                        mslt.at[pl.ds(w * CAP + f * SBC, SBC)])

        @pl.loop(0, CHK, step=16)
        def _(j):
          sts[pl.ds(j, 16)] = sts[pl.ds(SBC + j, 16)]
          stq[pl.ds(j, 16)] = stq[pl.ds(SBC + j, 16)]
        return o - SBC, f + 1

      def keep(o, f):
        return o, f

      return lax.cond(o >= SBC, flush, keep, o, f)

    o, f = lax.fori_loop(0, NCHK, chunk, (jnp.int32(0), jnp.int32(0)))
    pltpu.sync_copy(sts.at[pl.ds(0, SBC)],
                    msrc.at[pl.ds(w * CAP + f * SBC, SBC)])
    pltpu.sync_copy(stq.at[pl.ds(0, SBC)],
                    mslt.at[pl.ds(w * CAP + f * SBC, SBC)])
    tot = f * SBC + o
    cbuf[pl.ds(0, 16)] = jnp.where(lane0, tot, 0)
    pltpu.sync_copy(cbuf, mcnt.at[pl.ds(w * 16, 16)])

  @functools.partial(
      pl.kernel,
      out_type=jax.ShapeDtypeStruct((NWAVES * WAVE * FP,), _f32),
      mesh=_mesh,
      compiler_params=_cp,
      scratch_types=[
          pltpu.VMEM((SLW * FP,), _f32),
          pltpu.VMEM((128,), _i32),
          pltpu.VMEM((128,), _i32),
          pltpu.VMEM((FP,), _f32),
          pltpu.VMEM((16,), _i32),
      ],
  )
  def _k_m1(msrc, mslt, mcnt, hfe, nsl, m1out, local, srb, sbb, frow, cbuf):
    cid = lax.axis_index("c")
    sid = lax.axis_index("s")
    w = cid * 16 + sid
    it = _iota16()
    onehot18 = jnp.where(it == 2, 1.0, 0.0).astype(_f32)

    pltpu.sync_copy(nsl, cbuf)
    nstot = cbuf[pl.ds(0, 16)][0]

    for wave in range(NWAVES):
      tlo = wave * WAVE + w * SLW

      @pl.when(tlo < nstot)
      def _(tlo=tlo):
        @pl.loop(0, SLW * FP, step=16)
        def _(i):
          local[pl.ds(i, 16)] = jnp.zeros((16,), _f32)

        def region(r, _, tlo=tlo):
          pltpu.sync_copy(mcnt.at[pl.ds(r * 16, 16)], cbuf)
          tot = cbuf[pl.ds(0, 16)][0]
          nb = lax.shift_right_arithmetic(tot + 127, 7)

          def block(b, _, tot=tot, tlo=tlo):
            pltpu.sync_copy(msrc.at[pl.ds(r * CAP + b * 128, 128)], srb)
            pltpu.sync_copy(mslt.at[pl.ds(r * CAP + b * 128, 128)], sbb)
            rem = jnp.minimum(tot - b * 128, 128)

            def entry(e, _, tlo=tlo):
              qspl = plsc.load_gather(sbb, [_full16(e)])
              q = qspl[0]
              owned = jnp.logical_and(q >= tlo, q < tlo + SLW)

              @pl.when(owned)
              def _():
                s = plsc.load_gather(srb, [_full16(e)])[0]
                pltpu.sync_copy(hfe.at[pl.ds(s * FP, FP)], frow)
                dv = _rsqrt_deg(plsc.load_gather(frow, [_full16(FP - 1)]))
                lr = (q - tlo) * FP
                local[pl.ds(lr, 16)] = (local[pl.ds(lr, 16)] +
                                        frow[pl.ds(0, 16)] * dv)
                hi = frow[pl.ds(16, 16)] * dv + onehot18
                local[pl.ds(lr + 16, 16)] = local[pl.ds(lr + 16, 16)] + hi
              return 0

            lax.fori_loop(0, rem, entry, 0)
            return 0

          lax.fori_loop(0, nb, block, 0)
          return 0

        lax.fori_loop(0, NW, region, 0)

        pltpu.sync_copy(local, m1out.at[pl.ds(tlo * FP, SLW * FP)])

  @functools.partial(
      pl.kernel,
      out_type=jax.ShapeDtypeStruct((NW * 64,), _f32),
      mesh=_mesh,
      compiler_params=_cp,
      scratch_types=[
          pltpu.VMEM((F * 32,), _f32),
          pltpu.VMEM((32,), _f32),
          pltpu.VMEM((128,), _i32),
          pltpu.VMEM((128,), _i32),
          pltpu.VMEM((FP,), _f32),
          pltpu.VMEM((FP,), _f32),
          pltpu.VMEM((64,), _f32),
          pltpu.VMEM((16,), _i32),
      ],
  )
  def _k_m2(l0, l1, slt0, slt1, cnts, m1out, hfe, W1, b1,
            part, W1v, b1v, lb, qb, mrow, frow, pbuf, cbuf):
    cid = lax.axis_index("c")
    sid = lax.axis_index("s")
    w = cid * 16 + sid
    zero16 = jnp.zeros((16,), _f32)

    pltpu.sync_copy(W1, W1v)
    pltpu.sync_copy(b1, b1v)
    pltpu.sync_copy(cnts.at[pl.ds(w * 16, 16)], cbuf)
    cnt_v = cbuf[pl.ds(0, 16)]

    def process(lst, slst, tot):
      nb = lax.shift_right_arithmetic(tot + 127, 7)

      def block(b, accs, lst=lst, slst=slst, tot=tot):
        pltpu.sync_copy(lst.at[pl.ds(w * CAP + b * 128, 128)], lb)
        pltpu.sync_copy(slst.at[pl.ds(w * CAP + b * 128, 128)], qb)
        rem = jnp.minimum(tot - b * 128, 128)

        def edge(e, accs):
          a0, a1 = accs
          ef = _full16(e)
          s = plsc.load_gather(lb, [ef])[0]
          q = plsc.load_gather(qb, [ef])[0]
          pltpu.sync_copy(m1out.at[pl.ds(q * FP, FP)], mrow)
          pltpu.sync_copy(hfe.at[pl.ds(s * FP, FP)], frow)
          do = _rsqrt_deg(plsc.load_gather(frow, [_full16(FP - 1)]))
          di = _rsqrt_deg(plsc.load_gather(mrow, [_full16(F)]))
          mlo = zero16
          mhi = zero16
          for d in range(F):
            x = plsc.load_gather(mrow, [_full16(d)])
            mlo = mlo + x * W1v[pl.ds(d * 32, 16)]
            mhi = mhi + x * W1v[pl.ds(d * 32 + 16, 16)]
          h_lo = jnp.maximum(di * mlo + b1v[pl.ds(0, 16)], 0.0)
          h_hi = jnp.maximum(di * mhi + b1v[pl.ds(16, 16)], 0.0)
          return a0 + do * h_lo, a1 + do * h_hi

        return lax.fori_loop(0, rem, edge, accs)

      return lax.fori_loop(0, nb, block, (zero16, zero16))

    a00, a01 = process(l0, slt0, cnt_v[0])
    a10, a11 = process(l1, slt1, cnt_v[1])
    pbuf[pl.ds(0, 16)] = a00
    pbuf[pl.ds(16, 16)] = a01
    pbuf[pl.ds(32, 16)] = a10
    pbuf[pl.ds(48, 16)] = a11
    pltpu.sync_copy(pbuf, part.at[pl.ds(w * 64, 64)])

  @functools.partial(
      pl.kernel,
      out_type=(jax.ShapeDtypeStruct((16,), _f32),
                jax.ShapeDtypeStruct((2, 64), _f32),
                jax.ShapeDtypeStruct((2, 64), _f32)),
      mesh=_mesh,
      compiler_params=_cp,
      scratch_types=[
          pltpu.VMEM((NW * 64,), _f32),
          pltpu.VMEM((NW * 16,), _i32),
          pltpu.VMEM((32 * 16,), _f32),
          pltpu.VMEM((16,), _f32),
          pltpu.VMEM((16 * 64,), _f32),
          pltpu.VMEM((64,), _f32),
          pltpu.VMEM((64 * 64,), _f32),
          pltpu.VMEM((64,), _f32),
          pltpu.VMEM((64 * 64,), _f32),
          pltpu.VMEM((64,), _f32),
          pltpu.VMEM((32,), _f32),
          pltpu.VMEM((32 * 64,), _f32),
          pltpu.VMEM((64,), _f32),
          pltpu.VMEM((32,), _f32),
          pltpu.VMEM((16,), _f32),
          pltpu.VMEM((64,), _f32),
          pltpu.VMEM((64,), _f32),
          pltpu.VMEM((64,), _f32),
          pltpu.VMEM((64,), _f32),
          pltpu.VMEM((64,), _f32),
          pltpu.VMEM((2, 64), _f32),
          pltpu.VMEM((16,), _f32),
      ],
  )
  def _k_fin(part, cnts, W2, b2, Wp1, bp1, Wp2, bp2, Wp3, bp3, beta, Wq2, bq2,
             lout, z1o, z2o,
             pv, cv, W2v, b2v, Wp1v, bp1v, Wp2v, bp2v, Wp3v, bp3v,
             betav, Wq2v, bq2v, mbuf, hbuf, t1b, t2b, y0b, y1b, pb, zb, lb):
    cid = lax.axis_index("c")
    sid = lax.axis_index("s")

    @pl.when(jnp.logical_and(cid == 0, sid == 0))
    def _():
      pltpu.sync_copy(part, pv)
      pltpu.sync_copy(cnts, cv)
      pltpu.sync_copy(W2, W2v)
      pltpu.sync_copy(b2, b2v)
      pltpu.sync_copy(Wp1, Wp1v)
      pltpu.sync_copy(bp1, bp1v)
      pltpu.sync_copy(Wp2, Wp2v)
      pltpu.sync_copy(bp2, bp2v)
      pltpu.sync_copy(Wp3, Wp3v)
      pltpu.sync_copy(bp3, bp3v)
      pltpu.sync_copy(beta, betav)
      pltpu.sync_copy(Wq2, Wq2v)
      pltpu.sync_copy(bq2, bq2v)

      def matvec_to(xref, wref, nin, nout, bref, dst, relu):
        nacc = nout // 16

        def body(d, accs):
          xs = plsc.load_gather(xref, [_full16(d)])
          return tuple(accs[k] + xs * wref[pl.ds(d * nout + k * 16, 16)]
                       for k in range(nacc))

        accs = lax.fori_loop(
            0, nin, body,
            tuple(jnp.zeros((16,), _f32) for _ in range(nacc)))
        for k in range(nacc):
          v = accs[k] + bref[pl.ds(k * 16, 16)]
          if relu:
            v = jnp.maximum(v, 0.0)
          dst[pl.ds(k * 16, 16)] = v

      def accw(w, accs):
        return tuple(accs[k] + pv[pl.ds(w * 64 + k * 16, 16)]
                     for k in range(4))

      a = lax.fori_loop(0, NW, accw,
                        tuple(jnp.zeros((16,), _f32) for _ in range(4)))
      cacc = lax.fori_loop(0, NW,
                           lambda w, c: c + cv[pl.ds(w * 16, 16)],
                           jnp.zeros((16,), _i32))
      di0 = _rsqrt_deg(_full16(cacc[0]).astype(_f32))
      di1 = _rsqrt_deg(_full16(cacc[1]).astype(_f32))

      def project(lo, hi, di, ybuf):
        mbuf[pl.ds(0, 16)] = lo * di
        mbuf[pl.ds(16, 16)] = hi * di
        matvec_to(mbuf, W2v, 32, 16, b2v, hbuf, True)
        matvec_to(hbuf, Wp1v, 16, 64, bp1v, t1b, True)
        matvec_to(t1b, Wp2v, 64, 64, bp2v, t2b, True)
        matvec_to(t2b, Wp3v, 64, 64, bp3v, ybuf, False)

      project(a[0], a[1], di0, y0b)
      project(a[2], a[3], di1, y1b)

      betav[pl.ds(0, 16)] = jnp.maximum(betav[pl.ds(0, 16)], 0.0)
      betav[pl.ds(16, 16)] = jnp.maximum(betav[pl.ds(16, 16)], 0.0)
      matvec_to(betav, Wq2v, 32, 64, bq2v, pb, False)

      def dot64(aref, bref):
        s = jnp.zeros((16,), _f32)
        for k in range(4):
          s = s + aref[pl.ds(k * 16, 16)] * bref[pl.ds(k * 16, 16)]
        return _full16(jnp.sum(s), _f32)

      pn2 = dot64(pb, pb)

      def cos_with_p(yref):
        d = dot64(pb, yref)
        yn2 = dot64(yref, yref)
        prod = pn2 * yn2
        den = jnp.maximum(prod * _rsqrt_raw(prod), 1e-8)
        return d / den

      c0 = cos_with_p(y0b)
      c1 = cos_with_p(y1b)
      lb[pl.ds(0, 16)] = -0.5 * (c0 + c1)
      pltpu.sync_copy(lb, lout)

      for k in range(4):
        v = y0b[pl.ds(k * 16, 16)]
        zb[0, pl.ds(k * 16, 16)] = v
        zb[1, pl.ds(k * 16, 16)] = v
      pltpu.sync_copy(zb, z1o)
      for k in range(4):
        v = y1b[pl.ds(k * 16, 16)]
        zb[0, pl.ds(k * 16, 16)] = v
        zb[1, pl.ds(k * 16, 16)] = v
      pltpu.sync_copy(zb, z2o)

  return (_k_hist, _k_hred, _k_lists, _k_mslot, _k_m1, _k_m2, _k_fin)


def kernel(feats, edge_index, W1, b1, W2, b2, Wp1, bp1, Wp2, bp2, Wp3, bp3,
           Wq1, bq1, gamma, beta, Wq2, bq2):
  (_k_hist, _k_hred, _k_lists, _k_mslot, _k_m1, _k_m2, _k_fin) = _build()
  ei = edge_index.astype(jnp.int32)
  esrc = ei[0]
  edst = ei[1]
  feats = feats.astype(jnp.float32)
  hpart = _k_hist(esrc)
  dego = _k_hred(hpart)
  l0, l1, cnts = _k_lists(esrc, edst)
  msrc, mslt, mcnt, slt0, slt1, nsl = _k_mslot(esrc, edst, l0, l1, cnts)
  hfe = jnp.concatenate(
      [feats, jnp.zeros((N, FP - F - 1), jnp.float32), dego[:N, None]],
      axis=1).reshape(-1)
  m1out = _k_m1(msrc, mslt, mcnt, hfe, nsl)
  part = _k_m2(l0, l1, slt0, slt1, cnts, m1out, hfe,
               W1.reshape(-1), b1)
  lout, z1, z2 = _k_fin(part, cnts, W2.reshape(-1), b2, Wp1.reshape(-1), bp1,
                        Wp2.reshape(-1), bp2, Wp3.reshape(-1), bp3,
                        beta, Wq2.reshape(-1), bq2)
  return lout[0], z1, z2


# speedup vs baseline: 46.6673x; 1.9158x over previous
import dataclasses
import functools

import jax
import jax.numpy as jnp
from jax import lax
from jax.experimental import pallas as pl
from jax.experimental.pallas import tpu as pltpu
from jax.experimental.pallas import tpu_sc as plsc

N = 100000
E = 3200000
NP = 100352
NW = 32
EPW = E // NW
CHK = 2000
NCHK = EPW // CHK
SBC = 4096
STG = SBC + CHK
CAP = ((EPW // SBC) + 1) * SBC
F = 18
FP = 32
SLW = 2048
WAVE = NW * SLW
NWAVES = 2

_f32 = jnp.float32
_i32 = jnp.int32


def _iota16():
  return lax.iota(_i32, 16)


def _full16(x, dtype=_i32):
  return jnp.full((16,), x, dtype)


def _rsqrt_raw(x):
  i = plsc.bitcast(x, _i32)
  i = jnp.int32(0x5F3759DF) - lax.shift_right_arithmetic(i, 1)
  y = plsc.bitcast(i, _f32)
  for _ in range(3):
    y = y * (1.5 - 0.5 * x * y * y)
  return y


def _rsqrt_deg(x):
  return _rsqrt_raw(jnp.maximum(x, 1.0))


@functools.lru_cache(maxsize=1)
def _build():
  _mesh = plsc.VectorSubcoreMesh(core_axis_name="c", subcore_axis_name="s")
  _cp = pltpu.CompilerParams()
  if "needs_layout_passes" in pltpu.CompilerParams.__dataclass_fields__:
    _cp = dataclasses.replace(_cp, needs_layout_passes=False)

  @functools.partial(
      pl.kernel,
      out_type=jax.ShapeDtypeStruct((NW * NP,), _f32),
      mesh=_mesh,
      compiler_params=_cp,
      scratch_types=[
          pltpu.VMEM((NP,), _f32),
          pltpu.VMEM((CHK,), _i32),
      ],
  )
  def _k_hist(esrc, hpart, hist, ibuf):
    cid = lax.axis_index("c")
    sid = lax.axis_index("s")
    w = cid * 16 + sid

    @pl.loop(0, NP, step=16)
    def _(i):
      hist[pl.ds(i, 16)] = jnp.zeros((16,), _f32)

    @pl.loop(0, NCHK)
    def _(c):
      pltpu.sync_copy(esrc.at[pl.ds(w * EPW + c * CHK, CHK)], ibuf)

      @pl.loop(0, CHK, step=16)
      def _(j):
        v = ibuf[pl.ds(j, 16)]
        cnt, last = plsc.scan_count(v)
        plsc.addupdate_scatter(hist, [v], cnt.astype(_f32), mask=last)

    pltpu.sync_copy(hist, hpart.at[pl.ds(w * NP, NP)])

  @functools.partial(
      pl.kernel,
      out_type=jax.ShapeDtypeStruct((NP,), _f32),
      mesh=_mesh,
      compiler_params=_cp,
      scratch_types=[
          pltpu.VMEM((32, 1024), _f32),
          pltpu.VMEM((1024,), _f32),
      ],
  )
  def _k_hred(hpart, dego, rbuf, abuf):
    cid = lax.axis_index("c")
    sid = lax.axis_index("s")

    def emit(base, chunks):
      off = 0
      for ln in chunks:
        loff = off

        @pl.loop(0, NW)
        def _(p, loff=loff, ln=ln):
          pltpu.sync_copy(hpart.at[pl.ds(p * NP + base + loff, ln)],
                          rbuf.at[p, pl.ds(0, ln)])

        @pl.loop(0, ln, step=16)
        def _(j):
          acc = lax.fori_loop(
              0, NW, lambda i, acc: acc + rbuf[i, pl.ds(j, 16)],
              jnp.zeros((16,), _f32))
          abuf[pl.ds(j, 16)] = acc

        pltpu.sync_copy(abuf.at[pl.ds(0, ln)], dego.at[pl.ds(base + off, ln)])
        off += ln

    @pl.when(cid == 0)
    def _():
      emit(sid * 3200, [1024, 1024, 1024, 128])

    @pl.when(cid == 1)
    def _():
      emit(16 * 3200 + sid * 3072, [1024, 1024, 1024])

  @functools.partial(
      pl.kernel,
      out_type=(jax.ShapeDtypeStruct((NW * CAP,), _i32),
                jax.ShapeDtypeStruct((NW * CAP,), _i32),
                jax.ShapeDtypeStruct((NW * 16,), _i32)),
      mesh=_mesh,
      compiler_params=_cp,
      scratch_types=[
          pltpu.VMEM((CHK,), _i32),
          pltpu.VMEM((CHK,), _i32),
          pltpu.VMEM((STG,), _i32),
          pltpu.VMEM((STG,), _i32),
          pltpu.VMEM((16,), _i32),
      ],
  )
  def _k_lists(esrc, edst, l0, l1, cnts, ibs, ibd, st0, st1, cbuf):
    cid = lax.axis_index("c")
    sid = lax.axis_index("s")
    w = cid * 16 + sid

    def chunk(c, carry):
      o0, f0, o1, f1 = carry
      base = w * EPW + c * CHK
      pltpu.sync_copy(esrc.at[pl.ds(base, CHK)], ibs)
      pltpu.sync_copy(edst.at[pl.ds(base, CHK)], ibd)

      def vreg(j, carry):
        o0, o1 = carry
        vd = ibd[pl.ds(j * 16, 16)]
        vs = ibs[pl.ds(j * 16, 16)]
        m0 = vd == 0
        m1 = vd == 1
        plsc.store_compressed(st0.at[pl.ds(o0, 16)], vs, mask=m0)
        plsc.store_compressed(st1.at[pl.ds(o1, 16)], vs, mask=m1)
        o0 = o0 + plsc.all_reduce_population_count(m0)[0]
        o1 = o1 + plsc.all_reduce_population_count(m1)[0]
        return o0, o1

      o0, o1 = lax.fori_loop(0, CHK // 16, vreg, (o0, o1))

      def flush0(o, f):
        pltpu.sync_copy(st0.at[pl.ds(0, SBC)],
                        l0.at[pl.ds(w * CAP + f * SBC, SBC)])

        @pl.loop(0, CHK, step=16)
        def _(j):
          st0[pl.ds(j, 16)] = st0[pl.ds(SBC + j, 16)]
        return o - SBC, f + 1

      def flush1(o, f):
        pltpu.sync_copy(st1.at[pl.ds(0, SBC)],
                        l1.at[pl.ds(w * CAP + f * SBC, SBC)])

        @pl.loop(0, CHK, step=16)
        def _(j):
          st1[pl.ds(j, 16)] = st1[pl.ds(SBC + j, 16)]
        return o - SBC, f + 1

      def keep(o, f):
        return o, f

      o0, f0 = lax.cond(o0 >= SBC, flush0, keep, o0, f0)
      o1, f1 = lax.cond(o1 >= SBC, flush1, keep, o1, f1)
      return o0, f0, o1, f1

    z = jnp.int32(0)
    o0, f0, o1, f1 = lax.fori_loop(0, NCHK, chunk, (z, z, z, z))
    pltpu.sync_copy(st0.at[pl.ds(0, SBC)],
                    l0.at[pl.ds(w * CAP + f0 * SBC, SBC)])
    pltpu.sync_copy(st1.at[pl.ds(0, SBC)],
                    l1.at[pl.ds(w * CAP + f1 * SBC, SBC)])
    tot0 = f0 * SBC + o0
    tot1 = f1 * SBC + o1
    it = _iota16()
    cbuf[pl.ds(0, 16)] = jnp.where(it == 0, tot0,
                                   jnp.where(it == 1, tot1, 0))
    pltpu.sync_copy(cbuf, cnts.at[pl.ds(w * 16, 16)])

  @functools.partial(
      pl.kernel,
      out_type=(jax.ShapeDtypeStruct((NW * CAP,), _i32),
                jax.ShapeDtypeStruct((NW * CAP,), _i32),
                jax.ShapeDtypeStruct((NW * 16,), _i32),
                jax.ShapeDtypeStruct((NW * CAP,), _i32),
                jax.ShapeDtypeStruct((NW * CAP,), _i32),
                jax.ShapeDtypeStruct((16,), _i32)),
      mesh=_mesh,
      compiler_params=_cp,
      scratch_types=[
          pltpu.VMEM((NP,), _i32),
          pltpu.VMEM((CHK,), _i32),
          pltpu.VMEM((CHK,), _i32),
          pltpu.VMEM((STG,), _i32),
          pltpu.VMEM((STG,), _i32),
          pltpu.VMEM((128,), _i32),
          pltpu.VMEM((128,), _i32),
          pltpu.VMEM((NW * 16,), _i32),
          pltpu.VMEM((16,), _i32),
      ],
  )
  def _k_mslot(esrc, edst, l0, l1, cnts,
               msrc, mslt, mcnt, slt0, slt1, nsl,
               markv, ibs, ibd, sts, stq, lbuf, sbuf, cntv, cbuf):
    cid = lax.axis_index("c")
    sid = lax.axis_index("s")
    w = cid * 16 + sid
    it = _iota16()
    lane0 = it == 0

    @pl.loop(0, NP, step=16)
    def _(i):
      markv[pl.ds(i, 16)] = jnp.zeros((16,), _i32)

    pltpu.sync_copy(cnts, cntv)
    nxt = jnp.int32(1)
    for lst, slout, lane in ((l0, slt0, 0), (l1, slt1, 1)):
      def region(r, nxt, lst=lst, slout=slout, lane=lane):
        tot = plsc.load_gather(cntv, [_full16(r * 16 + lane)])[0]
        nb = lax.shift_right_arithmetic(tot + 127, 7)

        def block(b, nxt, lst=lst, slout=slout, tot=tot):
          pltpu.sync_copy(lst.at[pl.ds(r * CAP + b * 128, 128)], lbuf)
          rem = jnp.minimum(tot - b * 128, 128)

          def entry(e, nxt):
            sspl = plsc.load_gather(lbuf, [_full16(e)])
            old = plsc.load_gather(markv, [sspl])
            isnew = old == 0
            slotspl = jnp.where(isnew, _full16(nxt), old)
            plsc.store_scatter(markv, [sspl], slotspl,
                               mask=jnp.logical_and(isnew, lane0))

            @pl.when(w == 0)
            def _():
              plsc.store_scatter(sbuf, [_full16(e)], slotspl - 1, mask=lane0)
            return nxt + isnew.astype(_i32)[0]

          nxt = lax.fori_loop(0, rem, entry, nxt)

          @pl.when(w == 0)
          def _():
            pltpu.sync_copy(sbuf, slout.at[pl.ds(r * CAP + b * 128, 128)])
          return nxt

        return lax.fori_loop(0, nb, block, nxt)

      nxt = lax.fori_loop(0, NW, region, nxt)

    @pl.when(w == 0)
    def _():
      cbuf[pl.ds(0, 16)] = jnp.where(lane0, nxt - 1, 0)
      pltpu.sync_copy(cbuf, nsl)

    def chunk(c, carry):
      o, f = carry
      base = w * EPW + c * CHK
      pltpu.sync_copy(esrc.at[pl.ds(base, CHK)], ibs)
      pltpu.sync_copy(edst.at[pl.ds(base, CHK)], ibd)

      def vreg(j, o):
        vd = ibd[pl.ds(j * 16, 16)]
        vs = ibs[pl.ds(j * 16, 16)]
        g = plsc.load_gather(markv, [vd])
        m = g > 0
        plsc.store_compressed(sts.at[pl.ds(o, 16)], vs, mask=m)
        plsc.store_compressed(stq.at[pl.ds(o, 16)], g - 1, mask=m)
        return o + plsc.all_reduce_population_count(m)[0]

      o = lax.fori_loop(0, CHK // 16, vreg, o)

      def flush(o, f):
        pltpu.sync_copy(sts.at[pl.ds(0, SBC)],
                        msrc.at[pl.ds(w * CAP + f * SBC, SBC)])
        pltpu.sync_copy(stq.at[pl.ds(0, SBC)],
                        mslt.at[pl.ds(w * CAP + f * SBC, SBC)])

        @pl.loop(0, CHK, step=16)
        def _(j):
          sts[pl.ds(j, 16)] = sts[pl.ds(SBC + j, 16)]
          stq[pl.ds(j, 16)] = stq[pl.ds(SBC + j, 16)]
        return o - SBC, f + 1

      def keep(o, f):
        return o, f

      return lax.cond(o >= SBC, flush, keep, o, f)

    o, f = lax.fori_loop(0, NCHK, chunk, (jnp.int32(0), jnp.int32(0)))
    pltpu.sync_copy(sts.at[pl.ds(0, SBC)],
                    msrc.at[pl.ds(w * CAP + f * SBC, SBC)])
    pltpu.sync_copy(stq.at[pl.ds(0, SBC)],
                    mslt.at[pl.ds(w * CAP + f * SBC, SBC)])
    tot = f * SBC + o
    cbuf[pl.ds(0, 16)] = jnp.where(lane0, tot, 0)
    pltpu.sync_copy(cbuf, mcnt.at[pl.ds(w * 16, 16)])

  @functools.partial(
      pl.kernel,
      out_type=jax.ShapeDtypeStruct((NWAVES * WAVE * FP,), _f32),
      mesh=_mesh,
      compiler_params=_cp,
      scratch_types=[
          pltpu.VMEM((SLW * FP,), _f32),
          pltpu.VMEM((128,), _i32),
          pltpu.VMEM((128,), _i32),
          pltpu.VMEM((FP,), _f32),
          pltpu.VMEM((NW * 16,), _i32),
          pltpu.VMEM((16,), _i32),
      ],
  )
  def _k_m1(msrc, mslt, mcnt, hfe, nsl, m1out,
            local, srb, sbb, frow, cntv, cbuf):
    cid = lax.axis_index("c")
    sid = lax.axis_index("s")
    w = cid * 16 + sid
    it = _iota16()
    onehot18 = jnp.where(it == 2, 1.0, 0.0).astype(_f32)

    pltpu.sync_copy(nsl, cbuf)
    nstot = cbuf[pl.ds(0, 16)][0]
    pltpu.sync_copy(mcnt, cntv)

    for wave in range(NWAVES):
      @pl.when(wave * WAVE < nstot)
      def _(wave=wave):
        @pl.loop(0, SLW * FP, step=16)
        def _(i):
          local[pl.ds(i, 16)] = jnp.zeros((16,), _f32)

        def region(r, _, wave=wave):
          tot = plsc.load_gather(cntv, [_full16(r * 16)])[0]
          nb = lax.shift_right_arithmetic(tot + 127, 7)

          def block(b, _, tot=tot, wave=wave):
            pltpu.sync_copy(msrc.at[pl.ds(r * CAP + b * 128, 128)], srb)
            pltpu.sync_copy(mslt.at[pl.ds(r * CAP + b * 128, 128)], sbb)
            rem = jnp.minimum(tot - b * 128, 128)

            def entry(e, _, wave=wave):
              qspl = plsc.load_gather(sbb, [_full16(e)])
              q = qspl[0]
              owned = jnp.logical_and(
                  (q & 31) == w,
                  lax.shift_right_logical(q, 16) == wave)

              @pl.when(owned)
              def _():
                s = plsc.load_gather(srb, [_full16(e)])[0]
                pltpu.sync_copy(hfe.at[pl.ds(s * FP, FP)], frow)
                dv = _rsqrt_deg(plsc.load_gather(frow, [_full16(FP - 1)]))
                lr = (lax.shift_right_logical(q, 5) & (SLW - 1)) * FP
                local[pl.ds(lr, 16)] = (local[pl.ds(lr, 16)] +
                                        frow[pl.ds(0, 16)] * dv)
                hi = frow[pl.ds(16, 16)] * dv + onehot18
                local[pl.ds(lr + 16, 16)] = local[pl.ds(lr + 16, 16)] + hi
              return 0

            lax.fori_loop(0, rem, entry, 0)
            return 0

          lax.fori_loop(0, nb, block, 0)
          return 0

        lax.fori_loop(0, NW, region, 0)

        pltpu.sync_copy(
            local, m1out.at[pl.ds((wave * WAVE + w * SLW) * FP, SLW * FP)])

  @functools.partial(
      pl.kernel,
      out_type=jax.ShapeDtypeStruct((NW * 64,), _f32),
      mesh=_mesh,
      compiler_params=_cp,
      scratch_types=[
          pltpu.VMEM((F * 32,), _f32),
          pltpu.VMEM((32,), _f32),
          pltpu.VMEM((128,), _i32),
          pltpu.VMEM((128,), _i32),
          pltpu.VMEM((FP,), _f32),
          pltpu.VMEM((FP,), _f32),
          pltpu.VMEM((64,), _f32),
          pltpu.VMEM((16,), _i32),
      ],
  )
  def _k_m2(l0, l1, slt0, slt1, cnts, m1out, hfe, W1, b1,
            part, W1v, b1v, lb, qb, mrow, frow, pbuf, cbuf):
    cid = lax.axis_index("c")
    sid = lax.axis_index("s")
    w = cid * 16 + sid
    zero16 = jnp.zeros((16,), _f32)

    pltpu.sync_copy(W1, W1v)
    pltpu.sync_copy(b1, b1v)
    pltpu.sync_copy(cnts.at[pl.ds(w * 16, 16)], cbuf)
    cnt_v = cbuf[pl.ds(0, 16)]

    def process(lst, slst, tot):
      nb = lax.shift_right_arithmetic(tot + 127, 7)

      def block(b, accs, lst=lst, slst=slst, tot=tot):
        pltpu.sync_copy(lst.at[pl.ds(w * CAP + b * 128, 128)], lb)
        pltpu.sync_copy(slst.at[pl.ds(w * CAP + b * 128, 128)], qb)
        rem = jnp.minimum(tot - b * 128, 128)

        def edge(e, accs):
          a0, a1 = accs
          ef = _full16(e)
          s = plsc.load_gather(lb, [ef])[0]
          q = plsc.load_gather(qb, [ef])[0]
          prow = (lax.shift_right_logical(q, 16) * WAVE +
                  (q & 31) * SLW + (lax.shift_right_logical(q, 5) & (SLW - 1)))
          pltpu.sync_copy(m1out.at[pl.ds(prow * FP, FP)], mrow)
          pltpu.sync_copy(hfe.at[pl.ds(s * FP, FP)], frow)
          do = _rsqrt_deg(plsc.load_gather(frow, [_full16(FP - 1)]))
          di = _rsqrt_deg(plsc.load_gather(mrow, [_full16(F)]))
          mlo = zero16
          mhi = zero16
          for d in range(F):
            x = plsc.load_gather(mrow, [_full16(d)])
            mlo = mlo + x * W1v[pl.ds(d * 32, 16)]
            mhi = mhi + x * W1v[pl.ds(d * 32 + 16, 16)]
          h_lo = jnp.maximum(di * mlo + b1v[pl.ds(0, 16)], 0.0)
          h_hi = jnp.maximum(di * mhi + b1v[pl.ds(16, 16)], 0.0)
          return a0 + do * h_lo, a1 + do * h_hi

        return lax.fori_loop(0, rem, edge, accs)

      return lax.fori_loop(0, nb, block, (zero16, zero16))

    a00, a01 = process(l0, slt0, cnt_v[0])
    a10, a11 = process(l1, slt1, cnt_v[1])
    pbuf[pl.ds(0, 16)] = a00
    pbuf[pl.ds(16, 16)] = a01
    pbuf[pl.ds(32, 16)] = a10
    pbuf[pl.ds(48, 16)] = a11
    pltpu.sync_copy(pbuf, part.at[pl.ds(w * 64, 64)])

  @functools.partial(
      pl.kernel,
      out_type=(jax.ShapeDtypeStruct((16,), _f32),
                jax.ShapeDtypeStruct((2, 64), _f32),
                jax.ShapeDtypeStruct((2, 64), _f32)),
      mesh=_mesh,
      compiler_params=_cp,
      scratch_types=[
          pltpu.VMEM((NW * 64,), _f32),
          pltpu.VMEM((NW * 16,), _i32),
          pltpu.VMEM((32 * 16,), _f32),
          pltpu.VMEM((16,), _f32),
          pltpu.VMEM((16 * 64,), _f32),
          pltpu.VMEM((64,), _f32),
          pltpu.VMEM((64 * 64,), _f32),
          pltpu.VMEM((64,), _f32),
          pltpu.VMEM((64 * 64,), _f32),
          pltpu.VMEM((64,), _f32),
          pltpu.VMEM((32,), _f32),
          pltpu.VMEM((32 * 64,), _f32),
          pltpu.VMEM((64,), _f32),
          pltpu.VMEM((32,), _f32),
          pltpu.VMEM((16,), _f32),
          pltpu.VMEM((64,), _f32),
          pltpu.VMEM((64,), _f32),
          pltpu.VMEM((64,), _f32),
          pltpu.VMEM((64,), _f32),
          pltpu.VMEM((64,), _f32),
          pltpu.VMEM((2, 64), _f32),
          pltpu.VMEM((16,), _f32),
      ],
  )
  def _k_fin(part, cnts, W2, b2, Wp1, bp1, Wp2, bp2, Wp3, bp3, beta, Wq2, bq2,
             lout, z1o, z2o,
             pv, cv, W2v, b2v, Wp1v, bp1v, Wp2v, bp2v, Wp3v, bp3v,
             betav, Wq2v, bq2v, mbuf, hbuf, t1b, t2b, y0b, y1b, pb, zb, lb):
    cid = lax.axis_index("c")
    sid = lax.axis_index("s")

    @pl.when(jnp.logical_and(cid == 0, sid == 0))
    def _():
      pltpu.sync_copy(part, pv)
      pltpu.sync_copy(cnts, cv)
      pltpu.sync_copy(W2, W2v)
      pltpu.sync_copy(b2, b2v)
      pltpu.sync_copy(Wp1, Wp1v)
      pltpu.sync_copy(bp1, bp1v)
      pltpu.sync_copy(Wp2, Wp2v)
      pltpu.sync_copy(bp2, bp2v)
      pltpu.sync_copy(Wp3, Wp3v)
      pltpu.sync_copy(bp3, bp3v)
      pltpu.sync_copy(beta, betav)
      pltpu.sync_copy(Wq2, Wq2v)
      pltpu.sync_copy(bq2, bq2v)

      def matvec_to(xref, wref, nin, nout, bref, dst, relu):
        nacc = nout // 16

        def body(d, accs):
          xs = plsc.load_gather(xref, [_full16(d)])
          return tuple(accs[k] + xs * wref[pl.ds(d * nout + k * 16, 16)]
                       for k in range(nacc))

        accs = lax.fori_loop(
            0, nin, body,
            tuple(jnp.zeros((16,), _f32) for _ in range(nacc)))
        for k in range(nacc):
          v = accs[k] + bref[pl.ds(k * 16, 16)]
          if relu:
            v = jnp.maximum(v, 0.0)
          dst[pl.ds(k * 16, 16)] = v

      def accw(w, accs):
        return tuple(accs[k] + pv[pl.ds(w * 64 + k * 16, 16)]
                     for k in range(4))

      a = lax.fori_loop(0, NW, accw,
                        tuple(jnp.zeros((16,), _f32) for _ in range(4)))
      cacc = lax.fori_loop(0, NW,
                           lambda w, c: c + cv[pl.ds(w * 16, 16)],
                           jnp.zeros((16,), _i32))
      di0 = _rsqrt_deg(_full16(cacc[0]).astype(_f32))
      di1 = _rsqrt_deg(_full16(cacc[1]).astype(_f32))

      def project(lo, hi, di, ybuf):
        mbuf[pl.ds(0, 16)] = lo * di
        mbuf[pl.ds(16, 16)] = hi * di
        matvec_to(mbuf, W2v, 32, 16, b2v, hbuf, True)
        matvec_to(hbuf, Wp1v, 16, 64, bp1v, t1b, True)
        matvec_to(t1b, Wp2v, 64, 64, bp2v, t2b, True)
        matvec_to(t2b, Wp3v, 64, 64, bp3v, ybuf, False)

      project(a[0], a[1], di0, y0b)
      project(a[2], a[3], di1, y1b)

      betav[pl.ds(0, 16)] = jnp.maximum(betav[pl.ds(0, 16)], 0.0)
      betav[pl.ds(16, 16)] = jnp.maximum(betav[pl.ds(16, 16)], 0.0)
      matvec_to(betav, Wq2v, 32, 64, bq2v, pb, False)

      def dot64(aref, bref):
        s = jnp.zeros((16,), _f32)
        for k in range(4):
          s = s + aref[pl.ds(k * 16, 16)] * bref[pl.ds(k * 16, 16)]
        return _full16(jnp.sum(s), _f32)

      pn2 = dot64(pb, pb)

      def cos_with_p(yref):
        d = dot64(pb, yref)
        yn2 = dot64(yref, yref)
        prod = pn2 * yn2
        den = jnp.maximum(prod * _rsqrt_raw(prod), 1e-8)
        return d / den

      c0 = cos_with_p(y0b)
      c1 = cos_with_p(y1b)
      lb[pl.ds(0, 16)] = -0.5 * (c0 + c1)
      pltpu.sync_copy(lb, lout)

      for k in range(4):
        v = y0b[pl.ds(k * 16, 16)]
        zb[0, pl.ds(k * 16, 16)] = v
        zb[1, pl.ds(k * 16, 16)] = v
      pltpu.sync_copy(zb, z1o)
      for k in range(4):
        v = y1b[pl.ds(k * 16, 16)]
        zb[0, pl.ds(k * 16, 16)] = v
        zb[1, pl.ds(k * 16, 16)] = v
      pltpu.sync_copy(zb, z2o)

  return (_k_hist, _k_hred, _k_lists, _k_mslot, _k_m1, _k_m2, _k_fin)


def kernel(feats, edge_index, W1, b1, W2, b2, Wp1, bp1, Wp2, bp2, Wp3, bp3,
           Wq1, bq1, gamma, beta, Wq2, bq2):
  (_k_hist, _k_hred, _k_lists, _k_mslot, _k_m1, _k_m2, _k_fin) = _build()
  ei = edge_index.astype(jnp.int32)
  esrc = ei[0]
  edst = ei[1]
  feats = feats.astype(jnp.float32)
  hpart = _k_hist(esrc)
  dego = _k_hred(hpart)
  l0, l1, cnts = _k_lists(esrc, edst)
  msrc, mslt, mcnt, slt0, slt1, nsl = _k_mslot(esrc, edst, l0, l1, cnts)
  hfe = jnp.concatenate(
      [feats, jnp.zeros((N, FP - F - 1), jnp.float32), dego[:N, None]],
      axis=1).reshape(-1)
  m1out = _k_m1(msrc, mslt, mcnt, hfe, nsl)
  part = _k_m2(l0, l1, slt0, slt1, cnts, m1out, hfe,
               W1.reshape(-1), b1)
  lout, z1, z2 = _k_fin(part, cnts, W2.reshape(-1), b2, Wp1.reshape(-1), bp1,
                        Wp2.reshape(-1), bp2, Wp3.reshape(-1), bp3,
                        beta, Wq2.reshape(-1), bq2)
  return lout[0], z1, z2


# speedup vs baseline: 60.9314x; 1.3057x over previous
import dataclasses
import functools

import jax
import jax.numpy as jnp
from jax import lax
from jax.experimental import pallas as pl
from jax.experimental.pallas import tpu as pltpu
from jax.experimental.pallas import tpu_sc as plsc

N = 100000
E = 3200000
NP = 100352
NW = 32
EPW = E // NW
CHK = 2000
NCHK = EPW // CHK
SBC = 4096
STG = SBC + CHK
CAP = ((EPW // SBC) + 1) * SBC
F = 18
FP = 32
SLW = 2048
WAVE = NW * SLW
NWAVES = 2

_f32 = jnp.float32
_i32 = jnp.int32


def _iota16():
  return lax.iota(_i32, 16)


def _full16(x, dtype=_i32):
  return jnp.full((16,), x, dtype)


def _rsqrt_raw(x):
  i = plsc.bitcast(x, _i32)
  i = jnp.int32(0x5F3759DF) - lax.shift_right_arithmetic(i, 1)
  y = plsc.bitcast(i, _f32)
  for _ in range(3):
    y = y * (1.5 - 0.5 * x * y * y)
  return y


def _rsqrt_deg(x):
  return _rsqrt_raw(jnp.maximum(x, 1.0))


@functools.lru_cache(maxsize=1)
def _build():
  _mesh = plsc.VectorSubcoreMesh(core_axis_name="c", subcore_axis_name="s")
  _cp = pltpu.CompilerParams()
  if "needs_layout_passes" in pltpu.CompilerParams.__dataclass_fields__:
    _cp = dataclasses.replace(_cp, needs_layout_passes=False)

  @functools.partial(
      pl.kernel,
      out_type=jax.ShapeDtypeStruct((NW * NP,), _f32),
      mesh=_mesh,
      compiler_params=_cp,
      scratch_types=[
          pltpu.VMEM((NP,), _f32),
          pltpu.VMEM((CHK,), _i32),
          pltpu.VMEM((CHK,), _i32),
          pltpu.SemaphoreType.DMA,
          pltpu.SemaphoreType.DMA,
      ],
  )
  def _k_hist(esrc, hpart, hist, ibufa, ibufb, sema, semb):
    cid = lax.axis_index("c")
    sid = lax.axis_index("s")
    w = cid * 16 + sid

    @pl.loop(0, NP, step=16)
    def _(i):
      hist[pl.ds(i, 16)] = jnp.zeros((16,), _f32)

    def start(c, buf, sem):
      pltpu.async_copy(esrc.at[pl.ds(w * EPW + c * CHK, CHK)], buf, sem)

    def wait(buf, sem):
      pltpu.make_async_copy(esrc.at[pl.ds(0, CHK)], buf, sem).wait()

    def proc(buf):
      @pl.loop(0, CHK, step=16)
      def _(j):
        v = buf[pl.ds(j, 16)]
        cnt, last = plsc.scan_count(v)
        plsc.addupdate_scatter(hist, [v], cnt.astype(_f32), mask=last)

    start(0, ibufa, sema)
    start(1, ibufb, semb)

    def pair(q, _):
      wait(ibufa, sema)
      proc(ibufa)
      start(2 * q + 2, ibufa, sema)
      wait(ibufb, semb)
      proc(ibufb)
      start(2 * q + 3, ibufb, semb)
      return 0

    lax.fori_loop(0, (NCHK - 2) // 2, pair, 0)
    wait(ibufa, sema)
    proc(ibufa)
    wait(ibufb, semb)
    proc(ibufb)

    pltpu.sync_copy(hist, hpart.at[pl.ds(w * NP, NP)])

  @functools.partial(
      pl.kernel,
      out_type=jax.ShapeDtypeStruct((NP,), _f32),
      mesh=_mesh,
      compiler_params=_cp,
      scratch_types=[
          pltpu.VMEM((32, 1024), _f32),
          pltpu.VMEM((1024,), _f32),
          pltpu.SemaphoreType.DMA,
      ],
  )
  def _k_hred(hpart, dego, rbuf, abuf, sem):
    cid = lax.axis_index("c")
    sid = lax.axis_index("s")

    def emit(base, chunks):
      off = 0
      for ln in chunks:
        loff = off

        @pl.loop(0, NW)
        def _(p, loff=loff, ln=ln):
          pltpu.async_copy(hpart.at[pl.ds(p * NP + base + loff, ln)],
                           rbuf.at[p, pl.ds(0, ln)], sem)

        @pl.loop(0, NW)
        def _(p, loff=loff, ln=ln):
          pltpu.make_async_copy(hpart.at[pl.ds(base + loff, ln)],
                                rbuf.at[p, pl.ds(0, ln)], sem).wait()

        @pl.loop(0, ln, step=16)
        def _(j):
          acc = lax.fori_loop(
              0, NW, lambda i, acc: acc + rbuf[i, pl.ds(j, 16)],
              jnp.zeros((16,), _f32))
          abuf[pl.ds(j, 16)] = acc

        pltpu.sync_copy(abuf.at[pl.ds(0, ln)], dego.at[pl.ds(base + off, ln)])
        off += ln

    @pl.when(cid == 0)
    def _():
      emit(sid * 3200, [1024, 1024, 1024, 128])

    @pl.when(cid == 1)
    def _():
      emit(16 * 3200 + sid * 3072, [1024, 1024, 1024])

  @functools.partial(
      pl.kernel,
      out_type=(jax.ShapeDtypeStruct((NW * CAP,), _i32),
                jax.ShapeDtypeStruct((NW * CAP,), _i32),
                jax.ShapeDtypeStruct((NW * 16,), _i32)),
      mesh=_mesh,
      compiler_params=_cp,
      scratch_types=[
          pltpu.VMEM((CHK,), _i32),
          pltpu.VMEM((CHK,), _i32),
          pltpu.VMEM((CHK,), _i32),
          pltpu.VMEM((CHK,), _i32),
          pltpu.VMEM((STG,), _i32),
          pltpu.VMEM((STG,), _i32),
          pltpu.VMEM((16,), _i32),
          pltpu.SemaphoreType.DMA,
          pltpu.SemaphoreType.DMA,
      ],
  )
  def _k_lists(esrc, edst, l0, l1, cnts, ibsa, ibda, ibsb, ibdb,
               st0, st1, cbuf, sema, semb):
    cid = lax.axis_index("c")
    sid = lax.axis_index("s")
    w = cid * 16 + sid

    def start(c, bs, bd, sem):
      base = w * EPW + c * CHK
      pltpu.async_copy(esrc.at[pl.ds(base, CHK)], bs, sem)
      pltpu.async_copy(edst.at[pl.ds(base, CHK)], bd, sem)

    def wait(bs, bd, sem):
      pltpu.make_async_copy(esrc.at[pl.ds(0, CHK)], bs, sem).wait()
      pltpu.make_async_copy(edst.at[pl.ds(0, CHK)], bd, sem).wait()

    def chunk(ibs, ibd, carry):
      o0, f0, o1, f1 = carry

      def vreg(j, carry):
        o0, o1 = carry
        vd = ibd[pl.ds(j * 16, 16)]
        vs = ibs[pl.ds(j * 16, 16)]
        m0 = vd == 0
        m1 = vd == 1
        plsc.store_compressed(st0.at[pl.ds(o0, 16)], vs, mask=m0)
        plsc.store_compressed(st1.at[pl.ds(o1, 16)], vs, mask=m1)
        o0 = o0 + plsc.all_reduce_population_count(m0)[0]
        o1 = o1 + plsc.all_reduce_population_count(m1)[0]
        return o0, o1

      o0, o1 = lax.fori_loop(0, CHK // 16, vreg, (o0, o1))

      def flush0(o, f):
        pltpu.sync_copy(st0.at[pl.ds(0, SBC)],
                        l0.at[pl.ds(w * CAP + f * SBC, SBC)])

        @pl.loop(0, CHK, step=16)
        def _(j):
          st0[pl.ds(j, 16)] = st0[pl.ds(SBC + j, 16)]
        return o - SBC, f + 1

      def flush1(o, f):
        pltpu.sync_copy(st1.at[pl.ds(0, SBC)],
                        l1.at[pl.ds(w * CAP + f * SBC, SBC)])

        @pl.loop(0, CHK, step=16)
        def _(j):
          st1[pl.ds(j, 16)] = st1[pl.ds(SBC + j, 16)]
        return o - SBC, f + 1

      def keep(o, f):
        return o, f

      o0, f0 = lax.cond(o0 >= SBC, flush0, keep, o0, f0)
      o1, f1 = lax.cond(o1 >= SBC, flush1, keep, o1, f1)
      return o0, f0, o1, f1

    z = jnp.int32(0)
    start(0, ibsa, ibda, sema)
    start(1, ibsb, ibdb, semb)

    def pair(q, carry):
      wait(ibsa, ibda, sema)
      carry = chunk(ibsa, ibda, carry)
      start(2 * q + 2, ibsa, ibda, sema)
      wait(ibsb, ibdb, semb)
      carry = chunk(ibsb, ibdb, carry)
      start(2 * q + 3, ibsb, ibdb, semb)
      return carry

    carry = lax.fori_loop(0, (NCHK - 2) // 2, pair, (z, z, z, z))
    wait(ibsa, ibda, sema)
    carry = chunk(ibsa, ibda, carry)
    wait(ibsb, ibdb, semb)
    o0, f0, o1, f1 = chunk(ibsb, ibdb, carry)
    pltpu.sync_copy(st0.at[pl.ds(0, SBC)],
                    l0.at[pl.ds(w * CAP + f0 * SBC, SBC)])
    pltpu.sync_copy(st1.at[pl.ds(0, SBC)],
                    l1.at[pl.ds(w * CAP + f1 * SBC, SBC)])
    tot0 = f0 * SBC + o0
    tot1 = f1 * SBC + o1
    it = _iota16()
    cbuf[pl.ds(0, 16)] = jnp.where(it == 0, tot0,
                                   jnp.where(it == 1, tot1, 0))
    pltpu.sync_copy(cbuf, cnts.at[pl.ds(w * 16, 16)])

  @functools.partial(
      pl.kernel,
      out_type=(jax.ShapeDtypeStruct((NW * CAP,), _i32),
                jax.ShapeDtypeStruct((NW * CAP,), _i32),
                jax.ShapeDtypeStruct((NW * 16,), _i32),
                jax.ShapeDtypeStruct((NW * CAP,), _i32),
                jax.ShapeDtypeStruct((NW * CAP,), _i32),
                jax.ShapeDtypeStruct((16,), _i32)),
      mesh=_mesh,
      compiler_params=_cp,
      scratch_types=[
          pltpu.VMEM((NP,), _i32),
          pltpu.VMEM((CHK,), _i32),
          pltpu.VMEM((CHK,), _i32),
          pltpu.VMEM((CHK,), _i32),
          pltpu.VMEM((CHK,), _i32),
          pltpu.VMEM((STG,), _i32),
          pltpu.VMEM((STG,), _i32),
          pltpu.VMEM((128,), _i32),
          pltpu.VMEM((128,), _i32),
          pltpu.VMEM((NW * 16,), _i32),
          pltpu.VMEM((16,), _i32),
          pltpu.SemaphoreType.DMA,
          pltpu.SemaphoreType.DMA,
      ],
  )
  def _k_mslot(esrc, edst, l0, l1, cnts,
               msrc, mslt, mcnt, slt0, slt1, nsl,
               markv, ibs, ibd, ibsb, ibdb, sts, stq, lbuf, sbuf, cntv, cbuf,
               sema, semb):
    cid = lax.axis_index("c")
    sid = lax.axis_index("s")
    w = cid * 16 + sid
    it = _iota16()
    lane0 = it == 0

    @pl.loop(0, NP, step=16)
    def _(i):
      markv[pl.ds(i, 16)] = jnp.zeros((16,), _i32)

    pltpu.sync_copy(cnts, cntv)
    nxt = jnp.int32(1)
    for lst, slout, lane in ((l0, slt0, 0), (l1, slt1, 1)):
      def region(r, nxt, lst=lst, slout=slout, lane=lane):
        tot = plsc.load_gather(cntv, [_full16(r * 16 + lane)])[0]
        nb = lax.shift_right_arithmetic(tot + 127, 7)

        def block(b, nxt, lst=lst, slout=slout, tot=tot):
          pltpu.sync_copy(lst.at[pl.ds(r * CAP + b * 128, 128)], lbuf)
          rem = jnp.minimum(tot - b * 128, 128)

          def entry(e, nxt):
            sspl = plsc.load_gather(lbuf, [_full16(e)])
            old = plsc.load_gather(markv, [sspl])
            isnew = old == 0
            slotspl = jnp.where(isnew, _full16(nxt), old)
            plsc.store_scatter(markv, [sspl], slotspl,
                               mask=jnp.logical_and(isnew, lane0))

            @pl.when(w == 0)
            def _():
              plsc.store_scatter(sbuf, [_full16(e)], slotspl - 1, mask=lane0)
            return nxt + isnew.astype(_i32)[0]

          nxt = lax.fori_loop(0, rem, entry, nxt)

          @pl.when(w == 0)
          def _():
            pltpu.sync_copy(sbuf, slout.at[pl.ds(r * CAP + b * 128, 128)])
          return nxt

        return lax.fori_loop(0, nb, block, nxt)

      nxt = lax.fori_loop(0, NW, region, nxt)

    @pl.when(w == 0)
    def _():
      cbuf[pl.ds(0, 16)] = jnp.where(lane0, nxt - 1, 0)
      pltpu.sync_copy(cbuf, nsl)

    def start(c, bs, bd, sem):
      base = w * EPW + c * CHK
      pltpu.async_copy(esrc.at[pl.ds(base, CHK)], bs, sem)
      pltpu.async_copy(edst.at[pl.ds(base, CHK)], bd, sem)

    def wait(bs, bd, sem):
      pltpu.make_async_copy(esrc.at[pl.ds(0, CHK)], bs, sem).wait()
      pltpu.make_async_copy(edst.at[pl.ds(0, CHK)], bd, sem).wait()

    def chunk(ibs, ibd, carry):
      o, f = carry

      def vreg(j, o):
        vd = ibd[pl.ds(j * 16, 16)]
        vs = ibs[pl.ds(j * 16, 16)]
        g = plsc.load_gather(markv, [vd])
        m = g > 0
        plsc.store_compressed(sts.at[pl.ds(o, 16)], vs, mask=m)
        plsc.store_compressed(stq.at[pl.ds(o, 16)], g - 1, mask=m)
        return o + plsc.all_reduce_population_count(m)[0]

      o = lax.fori_loop(0, CHK // 16, vreg, o)

      def flush(o, f):
        pltpu.sync_copy(sts.at[pl.ds(0, SBC)],
                        msrc.at[pl.ds(w * CAP + f * SBC, SBC)])
        pltpu.sync_copy(stq.at[pl.ds(0, SBC)],
                        mslt.at[pl.ds(w * CAP + f * SBC, SBC)])

        @pl.loop(0, CHK, step=16)
        def _(j):
          sts[pl.ds(j, 16)] = sts[pl.ds(SBC + j, 16)]
          stq[pl.ds(j, 16)] = stq[pl.ds(SBC + j, 16)]
        return o - SBC, f + 1

      def keep(o, f):
        return o, f

      return lax.cond(o >= SBC, flush, keep, o, f)

    z2 = jnp.int32(0)
    start(0, ibs, ibd, sema)
    start(1, ibsb, ibdb, semb)

    def pair(q, carry):
      wait(ibs, ibd, sema)
      carry = chunk(ibs, ibd, carry)
      start(2 * q + 2, ibs, ibd, sema)
      wait(ibsb, ibdb, semb)
      carry = chunk(ibsb, ibdb, carry)
      start(2 * q + 3, ibsb, ibdb, semb)
      return carry

    carry = lax.fori_loop(0, (NCHK - 2) // 2, pair, (z2, z2))
    wait(ibs, ibd, sema)
    carry = chunk(ibs, ibd, carry)
    wait(ibsb, ibdb, semb)
    o, f = chunk(ibsb, ibdb, carry)
    pltpu.sync_copy(sts.at[pl.ds(0, SBC)],
                    msrc.at[pl.ds(w * CAP + f * SBC, SBC)])
    pltpu.sync_copy(stq.at[pl.ds(0, SBC)],
                    mslt.at[pl.ds(w * CAP + f * SBC, SBC)])
    tot = f * SBC + o
    cbuf[pl.ds(0, 16)] = jnp.where(lane0, tot, 0)
    pltpu.sync_copy(cbuf, mcnt.at[pl.ds(w * 16, 16)])

  @functools.partial(
      pl.kernel,
      out_type=jax.ShapeDtypeStruct((NWAVES * WAVE * FP,), _f32),
      mesh=_mesh,
      compiler_params=_cp,
      scratch_types=[
          pltpu.VMEM((SLW * FP,), _f32),
          pltpu.VMEM((128,), _i32),
          pltpu.VMEM((128,), _i32),
          pltpu.VMEM((FP,), _f32),
          pltpu.VMEM((NW * 16,), _i32),
          pltpu.VMEM((16,), _i32),
      ],
  )
  def _k_m1(msrc, mslt, mcnt, hfe, nsl, m1out,
            local, srb, sbb, frow, cntv, cbuf):
    cid = lax.axis_index("c")
    sid = lax.axis_index("s")
    w = cid * 16 + sid
    it = _iota16()
    onehot18 = jnp.where(it == 2, 1.0, 0.0).astype(_f32)

    pltpu.sync_copy(nsl, cbuf)
    nstot = cbuf[pl.ds(0, 16)][0]
    pltpu.sync_copy(mcnt, cntv)

    for wave in range(NWAVES):
      @pl.when(wave * WAVE < nstot)
      def _(wave=wave):
        @pl.loop(0, SLW * FP, step=16)
        def _(i):
          local[pl.ds(i, 16)] = jnp.zeros((16,), _f32)

        def region(r, _, wave=wave):
          tot = plsc.load_gather(cntv, [_full16(r * 16)])[0]
          nb = lax.shift_right_arithmetic(tot + 127, 7)

          def block(b, _, tot=tot, wave=wave):
            pltpu.sync_copy(msrc.at[pl.ds(r * CAP + b * 128, 128)], srb)
            pltpu.sync_copy(mslt.at[pl.ds(r * CAP + b * 128, 128)], sbb)
            rem = jnp.minimum(tot - b * 128, 128)

            def entry(e, _, wave=wave):
              qspl = plsc.load_gather(sbb, [_full16(e)])
              q = qspl[0]
              owned = jnp.logical_and(
                  (q & 31) == w,
                  lax.shift_right_logical(q, 16) == wave)

              @pl.when(owned)
              def _():
                s = plsc.load_gather(srb, [_full16(e)])[0]
                pltpu.sync_copy(hfe.at[pl.ds(s * FP, FP)], frow)
                dv = _rsqrt_deg(plsc.load_gather(frow, [_full16(FP - 1)]))
                lr = (lax.shift_right_logical(q, 5) & (SLW - 1)) * FP
                local[pl.ds(lr, 16)] = (local[pl.ds(lr, 16)] +
                                        frow[pl.ds(0, 16)] * dv)
                hi = frow[pl.ds(16, 16)] * dv + onehot18
                local[pl.ds(lr + 16, 16)] = local[pl.ds(lr + 16, 16)] + hi
              return 0

            lax.fori_loop(0, rem, entry, 0)
            return 0

          lax.fori_loop(0, nb, block, 0)
          return 0

        lax.fori_loop(0, NW, region, 0)

        pltpu.sync_copy(
            local, m1out.at[pl.ds((wave * WAVE + w * SLW) * FP, SLW * FP)])

  @functools.partial(
      pl.kernel,
      out_type=jax.ShapeDtypeStruct((NW * 64,), _f32),
      mesh=_mesh,
      compiler_params=_cp,
      scratch_types=[
          pltpu.VMEM((F * 32,), _f32),
          pltpu.VMEM((32,), _f32),
          pltpu.VMEM((128,), _i32),
          pltpu.VMEM((128,), _i32),
          pltpu.VMEM((FP,), _f32),
          pltpu.VMEM((FP,), _f32),
          pltpu.VMEM((64,), _f32),
          pltpu.VMEM((16,), _i32),
      ],
  )
  def _k_m2(l0, l1, slt0, slt1, cnts, m1out, hfe, W1, b1,
            part, W1v, b1v, lb, qb, mrow, frow, pbuf, cbuf):
    cid = lax.axis_index("c")
    sid = lax.axis_index("s")
    w = cid * 16 + sid
    zero16 = jnp.zeros((16,), _f32)

    pltpu.sync_copy(W1, W1v)
    pltpu.sync_copy(b1, b1v)
    pltpu.sync_copy(cnts.at[pl.ds(w * 16, 16)], cbuf)
    cnt_v = cbuf[pl.ds(0, 16)]

    def process(lst, slst, tot):
      nb = lax.shift_right_arithmetic(tot + 127, 7)

      def block(b, accs, lst=lst, slst=slst, tot=tot):
        pltpu.sync_copy(lst.at[pl.ds(w * CAP + b * 128, 128)], lb)
        pltpu.sync_copy(slst.at[pl.ds(w * CAP + b * 128, 128)], qb)
        rem = jnp.minimum(tot - b * 128, 128)

        def edge(e, accs):
          a0, a1 = accs
          ef = _full16(e)
          s = plsc.load_gather(lb, [ef])[0]
          q = plsc.load_gather(qb, [ef])[0]
          prow = (lax.shift_right_logical(q, 16) * WAVE +
                  (q & 31) * SLW + (lax.shift_right_logical(q, 5) & (SLW - 1)))
          pltpu.sync_copy(m1out.at[pl.ds(prow * FP, FP)], mrow)
          pltpu.sync_copy(hfe.at[pl.ds(s * FP, FP)], frow)
          do = _rsqrt_deg(plsc.load_gather(frow, [_full16(FP - 1)]))
          di = _rsqrt_deg(plsc.load_gather(mrow, [_full16(F)]))
          mlo = zero16
          mhi = zero16
          for d in range(F):
            x = plsc.load_gather(mrow, [_full16(d)])
            mlo = mlo + x * W1v[pl.ds(d * 32, 16)]
            mhi = mhi + x * W1v[pl.ds(d * 32 + 16, 16)]
          h_lo = jnp.maximum(di * mlo + b1v[pl.ds(0, 16)], 0.0)
          h_hi = jnp.maximum(di * mhi + b1v[pl.ds(16, 16)], 0.0)
          return a0 + do * h_lo, a1 + do * h_hi

        return lax.fori_loop(0, rem, edge, accs)

      return lax.fori_loop(0, nb, block, (zero16, zero16))

    a00, a01 = process(l0, slt0, cnt_v[0])
    a10, a11 = process(l1, slt1, cnt_v[1])
    pbuf[pl.ds(0, 16)] = a00
    pbuf[pl.ds(16, 16)] = a01
    pbuf[pl.ds(32, 16)] = a10
    pbuf[pl.ds(48, 16)] = a11
    pltpu.sync_copy(pbuf, part.at[pl.ds(w * 64, 64)])

  @functools.partial(
      pl.kernel,
      out_type=(jax.ShapeDtypeStruct((16,), _f32),
                jax.ShapeDtypeStruct((2, 64), _f32),
                jax.ShapeDtypeStruct((2, 64), _f32)),
      mesh=_mesh,
      compiler_params=_cp,
      scratch_types=[
          pltpu.VMEM((NW * 64,), _f32),
          pltpu.VMEM((NW * 16,), _i32),
          pltpu.VMEM((32 * 16,), _f32),
          pltpu.VMEM((16,), _f32),
          pltpu.VMEM((16 * 64,), _f32),
          pltpu.VMEM((64,), _f32),
          pltpu.VMEM((64 * 64,), _f32),
          pltpu.VMEM((64,), _f32),
          pltpu.VMEM((64 * 64,), _f32),
          pltpu.VMEM((64,), _f32),
          pltpu.VMEM((32,), _f32),
          pltpu.VMEM((32 * 64,), _f32),
          pltpu.VMEM((64,), _f32),
          pltpu.VMEM((32,), _f32),
          pltpu.VMEM((16,), _f32),
          pltpu.VMEM((64,), _f32),
          pltpu.VMEM((64,), _f32),
          pltpu.VMEM((64,), _f32),
          pltpu.VMEM((64,), _f32),
          pltpu.VMEM((64,), _f32),
          pltpu.VMEM((2, 64), _f32),
          pltpu.VMEM((16,), _f32),
      ],
  )
  def _k_fin(part, cnts, W2, b2, Wp1, bp1, Wp2, bp2, Wp3, bp3, beta, Wq2, bq2,
             lout, z1o, z2o,
             pv, cv, W2v, b2v, Wp1v, bp1v, Wp2v, bp2v, Wp3v, bp3v,
             betav, Wq2v, bq2v, mbuf, hbuf, t1b, t2b, y0b, y1b, pb, zb, lb):
    cid = lax.axis_index("c")
    sid = lax.axis_index("s")

    @pl.when(jnp.logical_and(cid == 0, sid == 0))
    def _():
      pltpu.sync_copy(part, pv)
      pltpu.sync_copy(cnts, cv)
      pltpu.sync_copy(W2, W2v)
      pltpu.sync_copy(b2, b2v)
      pltpu.sync_copy(Wp1, Wp1v)
      pltpu.sync_copy(bp1, bp1v)
      pltpu.sync_copy(Wp2, Wp2v)
      pltpu.sync_copy(bp2, bp2v)
      pltpu.sync_copy(Wp3, Wp3v)
      pltpu.sync_copy(bp3, bp3v)
      pltpu.sync_copy(beta, betav)
      pltpu.sync_copy(Wq2, Wq2v)
      pltpu.sync_copy(bq2, bq2v)

      def matvec_to(xref, wref, nin, nout, bref, dst, relu):
        nacc = nout // 16

        def body(d, accs):
          xs = plsc.load_gather(xref, [_full16(d)])
          return tuple(accs[k] + xs * wref[pl.ds(d * nout + k * 16, 16)]
                       for k in range(nacc))

        accs = lax.fori_loop(
            0, nin, body,
            tuple(jnp.zeros((16,), _f32) for _ in range(nacc)))
        for k in range(nacc):
          v = accs[k] + bref[pl.ds(k * 16, 16)]
          if relu:
            v = jnp.maximum(v, 0.0)
          dst[pl.ds(k * 16, 16)] = v

      def accw(w, accs):
        return tuple(accs[k] + pv[pl.ds(w * 64 + k * 16, 16)]
                     for k in range(4))

      a = lax.fori_loop(0, NW, accw,
                        tuple(jnp.zeros((16,), _f32) for _ in range(4)))
      cacc = lax.fori_loop(0, NW,
                           lambda w, c: c + cv[pl.ds(w * 16, 16)],
                           jnp.zeros((16,), _i32))
      di0 = _rsqrt_deg(_full16(cacc[0]).astype(_f32))
      di1 = _rsqrt_deg(_full16(cacc[1]).astype(_f32))

      def project(lo, hi, di, ybuf):
        mbuf[pl.ds(0, 16)] = lo * di
        mbuf[pl.ds(16, 16)] = hi * di
        matvec_to(mbuf, W2v, 32, 16, b2v, hbuf, True)
        matvec_to(hbuf, Wp1v, 16, 64, bp1v, t1b, True)
        matvec_to(t1b, Wp2v, 64, 64, bp2v, t2b, True)
        matvec_to(t2b, Wp3v, 64, 64, bp3v, ybuf, False)

      project(a[0], a[1], di0, y0b)
      project(a[2], a[3], di1, y1b)

      betav[pl.ds(0, 16)] = jnp.maximum(betav[pl.ds(0, 16)], 0.0)
      betav[pl.ds(16, 16)] = jnp.maximum(betav[pl.ds(16, 16)], 0.0)
      matvec_to(betav, Wq2v, 32, 64, bq2v, pb, False)

      def dot64(aref, bref):
        s = jnp.zeros((16,), _f32)
        for k in range(4):
          s = s + aref[pl.ds(k * 16, 16)] * bref[pl.ds(k * 16, 16)]
        return _full16(jnp.sum(s), _f32)

      pn2 = dot64(pb, pb)

      def cos_with_p(yref):
        d = dot64(pb, yref)
        yn2 = dot64(yref, yref)
        prod = pn2 * yn2
        den = jnp.maximum(prod * _rsqrt_raw(prod), 1e-8)
        return d / den

      c0 = cos_with_p(y0b)
      c1 = cos_with_p(y1b)
      lb[pl.ds(0, 16)] = -0.5 * (c0 + c1)
      pltpu.sync_copy(lb, lout)

      for k in range(4):
        v = y0b[pl.ds(k * 16, 16)]
        zb[0, pl.ds(k * 16, 16)] = v
        zb[1, pl.ds(k * 16, 16)] = v
      pltpu.sync_copy(zb, z1o)
      for k in range(4):
        v = y1b[pl.ds(k * 16, 16)]
        zb[0, pl.ds(k * 16, 16)] = v
        zb[1, pl.ds(k * 16, 16)] = v
      pltpu.sync_copy(zb, z2o)

  return (_k_hist, _k_hred, _k_lists, _k_mslot, _k_m1, _k_m2, _k_fin)


def kernel(feats, edge_index, W1, b1, W2, b2, Wp1, bp1, Wp2, bp2, Wp3, bp3,
           Wq1, bq1, gamma, beta, Wq2, bq2):
  (_k_hist, _k_hred, _k_lists, _k_mslot, _k_m1, _k_m2, _k_fin) = _build()
  ei = edge_index.astype(jnp.int32)
  esrc = ei[0]
  edst = ei[1]
  feats = feats.astype(jnp.float32)
  hpart = _k_hist(esrc)
  dego = _k_hred(hpart)
  l0, l1, cnts = _k_lists(esrc, edst)
  msrc, mslt, mcnt, slt0, slt1, nsl = _k_mslot(esrc, edst, l0, l1, cnts)
  hfe = jnp.concatenate(
      [feats, jnp.zeros((N, FP - F - 1), jnp.float32), dego[:N, None]],
      axis=1).reshape(-1)
  m1out = _k_m1(msrc, mslt, mcnt, hfe, nsl)
  part = _k_m2(l0, l1, slt0, slt1, cnts, m1out, hfe,
               W1.reshape(-1), b1)
  lout, z1, z2 = _k_fin(part, cnts, W2.reshape(-1), b2, Wp1.reshape(-1), bp1,
                        Wp2.reshape(-1), bp2, Wp3.reshape(-1), bp3,
                        beta, Wq2.reshape(-1), bq2)
  return lout[0], z1, z2


# speedup vs baseline: 67.3045x; 1.1046x over previous
import dataclasses
import functools

import jax
import jax.numpy as jnp
from jax import lax
from jax.experimental import pallas as pl
from jax.experimental.pallas import tpu as pltpu
from jax.experimental.pallas import tpu_sc as plsc

N = 100000
E = 3200000
NP = 100352
NW = 32
EPW = E // NW
CHK = 2000
NCHK = EPW // CHK
SBC = 4096
STG = SBC + CHK
CAP = ((EPW // SBC) + 1) * SBC
F = 18
FP = 32
SLW = 2048
WAVE = NW * SLW
NWAVES = 2

_f32 = jnp.float32
_i32 = jnp.int32


def _iota16():
  return lax.iota(_i32, 16)


def _full16(x, dtype=_i32):
  return jnp.full((16,), x, dtype)


def _rsqrt_raw(x):
  i = plsc.bitcast(x, _i32)
  i = jnp.int32(0x5F3759DF) - lax.shift_right_arithmetic(i, 1)
  y = plsc.bitcast(i, _f32)
  for _ in range(3):
    y = y * (1.5 - 0.5 * x * y * y)
  return y


def _rsqrt_deg(x):
  return _rsqrt_raw(jnp.maximum(x, 1.0))


@functools.lru_cache(maxsize=1)
def _build():
  _mesh = plsc.VectorSubcoreMesh(core_axis_name="c", subcore_axis_name="s")
  _cp = pltpu.CompilerParams()
  if "needs_layout_passes" in pltpu.CompilerParams.__dataclass_fields__:
    _cp = dataclasses.replace(_cp, needs_layout_passes=False)

  @functools.partial(
      pl.kernel,
      out_type=jax.ShapeDtypeStruct((NW * NP,), _f32),
      mesh=_mesh,
      compiler_params=_cp,
      scratch_types=[
          pltpu.VMEM((NP,), _f32),
          pltpu.VMEM((CHK,), _i32),
          pltpu.VMEM((CHK,), _i32),
          pltpu.SemaphoreType.DMA,
          pltpu.SemaphoreType.DMA,
      ],
  )
  def _k_hist(esrc, hpart, hist, ibufa, ibufb, sema, semb):
    cid = lax.axis_index("c")
    sid = lax.axis_index("s")
    w = cid * 16 + sid

    @pl.loop(0, NP, step=16)
    def _(i):
      hist[pl.ds(i, 16)] = jnp.zeros((16,), _f32)

    def start(c, buf, sem):
      pltpu.async_copy(esrc.at[pl.ds(w * EPW + c * CHK, CHK)], buf, sem)

    def wait(buf, sem):
      pltpu.make_async_copy(esrc.at[pl.ds(0, CHK)], buf, sem).wait()

    def proc(buf):
      @pl.loop(0, CHK, step=16)
      def _(j):
        v = buf[pl.ds(j, 16)]
        cnt, last = plsc.scan_count(v)
        plsc.addupdate_scatter(hist, [v], cnt.astype(_f32), mask=last)

    start(0, ibufa, sema)
    start(1, ibufb, semb)

    def pair(q, _):
      wait(ibufa, sema)
      proc(ibufa)
      start(2 * q + 2, ibufa, sema)
      wait(ibufb, semb)
      proc(ibufb)
      start(2 * q + 3, ibufb, semb)
      return 0

    lax.fori_loop(0, (NCHK - 2) // 2, pair, 0)
    wait(ibufa, sema)
    proc(ibufa)
    wait(ibufb, semb)
    proc(ibufb)

    pltpu.sync_copy(hist, hpart.at[pl.ds(w * NP, NP)])

  @functools.partial(
      pl.kernel,
      out_type=jax.ShapeDtypeStruct((NP,), _f32),
      mesh=_mesh,
      compiler_params=_cp,
      scratch_types=[
          pltpu.VMEM((32, 1024), _f32),
          pltpu.VMEM((1024,), _f32),
          pltpu.SemaphoreType.DMA,
      ],
  )
  def _k_hred(hpart, dego, rbuf, abuf, sem):
    cid = lax.axis_index("c")
    sid = lax.axis_index("s")

    def emit(base, chunks):
      off = 0
      for ln in chunks:
        loff = off

        @pl.loop(0, NW)
        def _(p, loff=loff, ln=ln):
          pltpu.async_copy(hpart.at[pl.ds(p * NP + base + loff, ln)],
                           rbuf.at[p, pl.ds(0, ln)], sem)

        @pl.loop(0, NW)
        def _(p, loff=loff, ln=ln):
          pltpu.make_async_copy(hpart.at[pl.ds(base + loff, ln)],
                                rbuf.at[p, pl.ds(0, ln)], sem).wait()

        @pl.loop(0, ln, step=16)
        def _(j):
          acc = lax.fori_loop(
              0, NW, lambda i, acc: acc + rbuf[i, pl.ds(j, 16)],
              jnp.zeros((16,), _f32))
          abuf[pl.ds(j, 16)] = acc

        pltpu.sync_copy(abuf.at[pl.ds(0, ln)], dego.at[pl.ds(base + off, ln)])
        off += ln

    @pl.when(cid == 0)
    def _():
      emit(sid * 3200, [1024, 1024, 1024, 128])

    @pl.when(cid == 1)
    def _():
      emit(16 * 3200 + sid * 3072, [1024, 1024, 1024])

  @functools.partial(
      pl.kernel,
      out_type=(jax.ShapeDtypeStruct((NW * CAP,), _i32),
                jax.ShapeDtypeStruct((NW * CAP,), _i32),
                jax.ShapeDtypeStruct((NW * 16,), _i32)),
      mesh=_mesh,
      compiler_params=_cp,
      scratch_types=[
          pltpu.VMEM((CHK,), _i32),
          pltpu.VMEM((CHK,), _i32),
          pltpu.VMEM((CHK,), _i32),
          pltpu.VMEM((CHK,), _i32),
          pltpu.VMEM((STG,), _i32),
          pltpu.VMEM((STG,), _i32),
          pltpu.VMEM((16,), _i32),
          pltpu.SemaphoreType.DMA,
          pltpu.SemaphoreType.DMA,
      ],
  )
  def _k_lists(esrc, edst, l0, l1, cnts, ibsa, ibda, ibsb, ibdb,
               st0, st1, cbuf, sema, semb):
    cid = lax.axis_index("c")
    sid = lax.axis_index("s")
    w = cid * 16 + sid

    def start(c, bs, bd, sem):
      base = w * EPW + c * CHK
      pltpu.async_copy(esrc.at[pl.ds(base, CHK)], bs, sem)
      pltpu.async_copy(edst.at[pl.ds(base, CHK)], bd, sem)

    def wait(bs, bd, sem):
      pltpu.make_async_copy(esrc.at[pl.ds(0, CHK)], bs, sem).wait()
      pltpu.make_async_copy(edst.at[pl.ds(0, CHK)], bd, sem).wait()

    def chunk(ibs, ibd, carry):
      o0, f0, o1, f1 = carry

      def vreg(j, carry):
        o0, o1 = carry
        vd = ibd[pl.ds(j * 16, 16)]
        vs = ibs[pl.ds(j * 16, 16)]
        m0 = vd == 0
        m1 = vd == 1
        plsc.store_compressed(st0.at[pl.ds(o0, 16)], vs, mask=m0)
        plsc.store_compressed(st1.at[pl.ds(o1, 16)], vs, mask=m1)
        o0 = o0 + plsc.all_reduce_population_count(m0)[0]
        o1 = o1 + plsc.all_reduce_population_count(m1)[0]
        return o0, o1

      o0, o1 = lax.fori_loop(0, CHK // 16, vreg, (o0, o1))

      def flush0(o, f):
        pltpu.sync_copy(st0.at[pl.ds(0, SBC)],
                        l0.at[pl.ds(w * CAP + f * SBC, SBC)])

        @pl.loop(0, CHK, step=16)
        def _(j):
          st0[pl.ds(j, 16)] = st0[pl.ds(SBC + j, 16)]
        return o - SBC, f + 1

      def flush1(o, f):
        pltpu.sync_copy(st1.at[pl.ds(0, SBC)],
                        l1.at[pl.ds(w * CAP + f * SBC, SBC)])

        @pl.loop(0, CHK, step=16)
        def _(j):
          st1[pl.ds(j, 16)] = st1[pl.ds(SBC + j, 16)]
        return o - SBC, f + 1

      def keep(o, f):
        return o, f

      o0, f0 = lax.cond(o0 >= SBC, flush0, keep, o0, f0)
      o1, f1 = lax.cond(o1 >= SBC, flush1, keep, o1, f1)
      return o0, f0, o1, f1

    z = jnp.int32(0)
    start(0, ibsa, ibda, sema)
    start(1, ibsb, ibdb, semb)

    def pair(q, carry):
      wait(ibsa, ibda, sema)
      carry = chunk(ibsa, ibda, carry)
      start(2 * q + 2, ibsa, ibda, sema)
      wait(ibsb, ibdb, semb)
      carry = chunk(ibsb, ibdb, carry)
      start(2 * q + 3, ibsb, ibdb, semb)
      return carry

    carry = lax.fori_loop(0, (NCHK - 2) // 2, pair, (z, z, z, z))
    wait(ibsa, ibda, sema)
    carry = chunk(ibsa, ibda, carry)
    wait(ibsb, ibdb, semb)
    o0, f0, o1, f1 = chunk(ibsb, ibdb, carry)
    pltpu.sync_copy(st0.at[pl.ds(0, SBC)],
                    l0.at[pl.ds(w * CAP + f0 * SBC, SBC)])
    pltpu.sync_copy(st1.at[pl.ds(0, SBC)],
                    l1.at[pl.ds(w * CAP + f1 * SBC, SBC)])
    tot0 = f0 * SBC + o0
    tot1 = f1 * SBC + o1
    it = _iota16()
    cbuf[pl.ds(0, 16)] = jnp.where(it == 0, tot0,
                                   jnp.where(it == 1, tot1, 0))
    pltpu.sync_copy(cbuf, cnts.at[pl.ds(w * 16, 16)])

  @functools.partial(
      pl.kernel,
      out_type=(jax.ShapeDtypeStruct((NW * CAP,), _i32),
                jax.ShapeDtypeStruct((NW * CAP,), _i32),
                jax.ShapeDtypeStruct((NW * 16,), _i32),
                jax.ShapeDtypeStruct((NW * CAP,), _i32),
                jax.ShapeDtypeStruct((NW * CAP,), _i32),
                jax.ShapeDtypeStruct((16,), _i32)),
      mesh=_mesh,
      compiler_params=_cp,
      scratch_types=[
          pltpu.VMEM((NP,), _i32),
          pltpu.VMEM((CHK,), _i32),
          pltpu.VMEM((CHK,), _i32),
          pltpu.VMEM((CHK,), _i32),
          pltpu.VMEM((CHK,), _i32),
          pltpu.VMEM((STG,), _i32),
          pltpu.VMEM((STG,), _i32),
          pltpu.VMEM((128,), _i32),
          pltpu.VMEM((128,), _i32),
          pltpu.VMEM((NW * 16,), _i32),
          pltpu.VMEM((16,), _i32),
          pltpu.SemaphoreType.DMA,
          pltpu.SemaphoreType.DMA,
      ],
  )
  def _k_mslot(esrc, edst, l0, l1, cnts,
               msrc, mslt, mcnt, slt0, slt1, nsl,
               markv, ibs, ibd, ibsb, ibdb, sts, stq, lbuf, sbuf, cntv, cbuf,
               sema, semb):
    cid = lax.axis_index("c")
    sid = lax.axis_index("s")
    w = cid * 16 + sid
    it = _iota16()
    lane0 = it == 0

    @pl.loop(0, NP, step=16)
    def _(i):
      markv[pl.ds(i, 16)] = jnp.zeros((16,), _i32)

    pltpu.sync_copy(cnts, cntv)
    nxt = jnp.int32(1)
    for lst, slout, lane in ((l0, slt0, 0), (l1, slt1, 1)):
      def region(r, nxt, lst=lst, slout=slout, lane=lane):
        tot = plsc.load_gather(cntv, [_full16(r * 16 + lane)])[0]
        nb = lax.shift_right_arithmetic(tot + 127, 7)

        def block(b, nxt, lst=lst, slout=slout, tot=tot):
          pltpu.sync_copy(lst.at[pl.ds(r * CAP + b * 128, 128)], lbuf)
          rem = jnp.minimum(tot - b * 128, 128)

          def entry(e, nxt):
            sspl = plsc.load_gather(lbuf, [_full16(e)])
            old = plsc.load_gather(markv, [sspl])
            isnew = old == 0
            slotspl = jnp.where(isnew, _full16(nxt), old)
            plsc.store_scatter(markv, [sspl], slotspl,
                               mask=jnp.logical_and(isnew, lane0))

            @pl.when(w == 0)
            def _():
              plsc.store_scatter(sbuf, [_full16(e)], slotspl - 1, mask=lane0)
            return nxt + isnew.astype(_i32)[0]

          nxt = lax.fori_loop(0, rem, entry, nxt)

          @pl.when(w == 0)
          def _():
            pltpu.sync_copy(sbuf, slout.at[pl.ds(r * CAP + b * 128, 128)])
          return nxt

        return lax.fori_loop(0, nb, block, nxt)

      nxt = lax.fori_loop(0, NW, region, nxt)

    @pl.when(w == 0)
    def _():
      cbuf[pl.ds(0, 16)] = jnp.where(lane0, nxt - 1, 0)
      pltpu.sync_copy(cbuf, nsl)

    def start(c, bs, bd, sem):
      base = w * EPW + c * CHK
      pltpu.async_copy(esrc.at[pl.ds(base, CHK)], bs, sem)
      pltpu.async_copy(edst.at[pl.ds(base, CHK)], bd, sem)

    def wait(bs, bd, sem):
      pltpu.make_async_copy(esrc.at[pl.ds(0, CHK)], bs, sem).wait()
      pltpu.make_async_copy(edst.at[pl.ds(0, CHK)], bd, sem).wait()

    def chunk(ibs, ibd, carry):
      o, f = carry

      def vreg(j, o):
        vd = ibd[pl.ds(j * 16, 16)]
        vs = ibs[pl.ds(j * 16, 16)]
        g = plsc.load_gather(markv, [vd])
        m = g > 0
        plsc.store_compressed(sts.at[pl.ds(o, 16)], vs, mask=m)
        plsc.store_compressed(stq.at[pl.ds(o, 16)], g - 1, mask=m)
        return o + plsc.all_reduce_population_count(m)[0]

      o = lax.fori_loop(0, CHK // 16, vreg, o)

      def flush(o, f):
        pltpu.sync_copy(sts.at[pl.ds(0, SBC)],
                        msrc.at[pl.ds(w * CAP + f * SBC, SBC)])
        pltpu.sync_copy(stq.at[pl.ds(0, SBC)],
                        mslt.at[pl.ds(w * CAP + f * SBC, SBC)])

        @pl.loop(0, CHK, step=16)
        def _(j):
          sts[pl.ds(j, 16)] = sts[pl.ds(SBC + j, 16)]
          stq[pl.ds(j, 16)] = stq[pl.ds(SBC + j, 16)]
        return o - SBC, f + 1

      def keep(o, f):
        return o, f

      return lax.cond(o >= SBC, flush, keep, o, f)

    z2 = jnp.int32(0)
    start(0, ibs, ibd, sema)
    start(1, ibsb, ibdb, semb)

    def pair(q, carry):
      wait(ibs, ibd, sema)
      carry = chunk(ibs, ibd, carry)
      start(2 * q + 2, ibs, ibd, sema)
      wait(ibsb, ibdb, semb)
      carry = chunk(ibsb, ibdb, carry)
      start(2 * q + 3, ibsb, ibdb, semb)
      return carry

    carry = lax.fori_loop(0, (NCHK - 2) // 2, pair, (z2, z2))
    wait(ibs, ibd, sema)
    carry = chunk(ibs, ibd, carry)
    wait(ibsb, ibdb, semb)
    o, f = chunk(ibsb, ibdb, carry)
    pltpu.sync_copy(sts.at[pl.ds(0, SBC)],
                    msrc.at[pl.ds(w * CAP + f * SBC, SBC)])
    pltpu.sync_copy(stq.at[pl.ds(0, SBC)],
                    mslt.at[pl.ds(w * CAP + f * SBC, SBC)])
    tot = f * SBC + o
    cbuf[pl.ds(0, 16)] = jnp.where(lane0, tot, 0)
    pltpu.sync_copy(cbuf, mcnt.at[pl.ds(w * 16, 16)])

  @functools.partial(
      pl.kernel,
      out_type=jax.ShapeDtypeStruct((NWAVES * WAVE * FP,), _f32),
      mesh=_mesh,
      compiler_params=_cp,
      scratch_types=[
          pltpu.VMEM((SLW * FP,), _f32),
          pltpu.VMEM((NW, 128), _i32),
          pltpu.VMEM((NW, 128), _i32),
          pltpu.VMEM((FP,), _f32),
          pltpu.VMEM((NW * 16,), _i32),
          pltpu.VMEM((16,), _i32),
          pltpu.SemaphoreType.DMA,
      ],
  )
  def _k_m1(msrc, mslt, mcnt, hfe, nsl, m1out,
            local, srbv, sbbv, frow, cntv, cbuf, sem):
    cid = lax.axis_index("c")
    sid = lax.axis_index("s")
    w = cid * 16 + sid
    it = _iota16()
    onehot18 = jnp.where(it == 2, 1.0, 0.0).astype(_f32)

    pltpu.sync_copy(nsl, cbuf)
    nstot = cbuf[pl.ds(0, 16)][0]
    pltpu.sync_copy(mcnt, cntv)
    maxtot = lax.fori_loop(
        0, NW,
        lambda r, m: jnp.maximum(m, plsc.load_gather(cntv,
                                                     [_full16(r * 16)])[0]),
        jnp.int32(0))
    maxnb = lax.shift_right_arithmetic(maxtot + 127, 7)

    for wave in range(NWAVES):
      @pl.when(wave * WAVE < nstot)
      def _(wave=wave):
        @pl.loop(0, SLW * FP, step=16)
        def _(i):
          local[pl.ds(i, 16)] = jnp.zeros((16,), _f32)

        def bround(b, _, wave=wave):
          def burst(g, _):
            def fire(r, _):
              tot = plsc.load_gather(cntv, [_full16(r * 16)])[0]
              nb1 = jnp.maximum(
                  lax.shift_right_arithmetic(tot + 127, 7) - 1, 0)
              bc = jnp.minimum(b, nb1)
              pltpu.async_copy(msrc.at[pl.ds(r * CAP + bc * 128, 128)],
                               srbv.at[r], sem)
              pltpu.async_copy(mslt.at[pl.ds(r * CAP + bc * 128, 128)],
                               sbbv.at[r], sem)
              return 0

            lax.fori_loop(g * 8, g * 8 + 8, fire, 0)

            def drain(r, _):
              pltpu.make_async_copy(msrc.at[pl.ds(0, 128)], srbv.at[r],
                                    sem).wait()
              pltpu.make_async_copy(mslt.at[pl.ds(0, 128)], sbbv.at[r],
                                    sem).wait()
              return 0

            lax.fori_loop(g * 8, g * 8 + 8, drain, 0)
            return 0

          lax.fori_loop(0, NW // 8, burst, 0)

          def region(r, _, wave=wave):
            tot = plsc.load_gather(cntv, [_full16(r * 16)])[0]

            def vchunk(k, _, wave=wave):
              qv = sbbv[r, pl.ds(k * 16, 16)]
              posv = b * 128 + k * 16 + it
              ownedv = jnp.logical_and(
                  jnp.logical_and((qv & 31) == w,
                                  lax.shift_right_logical(qv, 16) == wave),
                  posv < tot)
              cnt = plsc.all_reduce_population_count(ownedv)[0]

              @pl.when(cnt > 0)
              def _(wave=wave):
                def lane(e2, _, wave=wave):
                  ef = _full16(k * 16 + e2)
                  q = plsc.load_gather(sbbv, [_full16(r), ef])[0]
                  pos = b * 128 + k * 16 + e2
                  owned = jnp.logical_and(
                      jnp.logical_and((q & 31) == w,
                                      lax.shift_right_logical(q, 16) == wave),
                      pos < tot)

                  @pl.when(owned)
                  def _():
                    s = plsc.load_gather(srbv, [_full16(r), ef])[0]
                    pltpu.sync_copy(hfe.at[pl.ds(s * FP, FP)], frow)
                    dv = _rsqrt_deg(plsc.load_gather(frow,
                                                     [_full16(FP - 1)]))
                    lr = (lax.shift_right_logical(q, 5) & (SLW - 1)) * FP
                    local[pl.ds(lr, 16)] = (local[pl.ds(lr, 16)] +
                                            frow[pl.ds(0, 16)] * dv)
                    hi = frow[pl.ds(16, 16)] * dv + onehot18
                    local[pl.ds(lr + 16, 16)] = (local[pl.ds(lr + 16, 16)] +
                                                 hi)
                  return 0

                lax.fori_loop(0, 16, lane, 0)
              return 0

            lax.fori_loop(0, 8, vchunk, 0)
            return 0

          lax.fori_loop(0, NW, region, 0)
          return 0

        lax.fori_loop(0, maxnb, bround, 0)

        pltpu.sync_copy(
            local, m1out.at[pl.ds((wave * WAVE + w * SLW) * FP, SLW * FP)])

  @functools.partial(
      pl.kernel,
      out_type=jax.ShapeDtypeStruct((NW * 64,), _f32),
      mesh=_mesh,
      compiler_params=_cp,
      scratch_types=[
          pltpu.VMEM((F * 32,), _f32),
          pltpu.VMEM((32,), _f32),
          pltpu.VMEM((128,), _i32),
          pltpu.VMEM((128,), _i32),
          pltpu.VMEM((FP,), _f32),
          pltpu.VMEM((FP,), _f32),
          pltpu.VMEM((64,), _f32),
          pltpu.VMEM((16,), _i32),
      ],
  )
  def _k_m2(l0, l1, slt0, slt1, cnts, m1out, hfe, W1, b1,
            part, W1v, b1v, lb, qb, mrow, frow, pbuf, cbuf):
    cid = lax.axis_index("c")
    sid = lax.axis_index("s")
    w = cid * 16 + sid
    zero16 = jnp.zeros((16,), _f32)

    pltpu.sync_copy(W1, W1v)
    pltpu.sync_copy(b1, b1v)
    pltpu.sync_copy(cnts.at[pl.ds(w * 16, 16)], cbuf)
    cnt_v = cbuf[pl.ds(0, 16)]

    def process(lst, slst, tot):
      nb = lax.shift_right_arithmetic(tot + 127, 7)

      def block(b, accs, lst=lst, slst=slst, tot=tot):
        pltpu.sync_copy(lst.at[pl.ds(w * CAP + b * 128, 128)], lb)
        pltpu.sync_copy(slst.at[pl.ds(w * CAP + b * 128, 128)], qb)
        rem = jnp.minimum(tot - b * 128, 128)

        def edge(e, accs):
          a0, a1 = accs
          ef = _full16(e)
          s = plsc.load_gather(lb, [ef])[0]
          q = plsc.load_gather(qb, [ef])[0]
          prow = (lax.shift_right_logical(q, 16) * WAVE +
                  (q & 31) * SLW + (lax.shift_right_logical(q, 5) & (SLW - 1)))
          pltpu.sync_copy(m1out.at[pl.ds(prow * FP, FP)], mrow)
          pltpu.sync_copy(hfe.at[pl.ds(s * FP, FP)], frow)
          do = _rsqrt_deg(plsc.load_gather(frow, [_full16(FP - 1)]))
          di = _rsqrt_deg(plsc.load_gather(mrow, [_full16(F)]))
          mlo = zero16
          mhi = zero16
          for d in range(F):
            x = plsc.load_gather(mrow, [_full16(d)])
            mlo = mlo + x * W1v[pl.ds(d * 32, 16)]
            mhi = mhi + x * W1v[pl.ds(d * 32 + 16, 16)]
          h_lo = jnp.maximum(di * mlo + b1v[pl.ds(0, 16)], 0.0)
          h_hi = jnp.maximum(di * mhi + b1v[pl.ds(16, 16)], 0.0)
          return a0 + do * h_lo, a1 + do * h_hi

        return lax.fori_loop(0, rem, edge, accs)

      return lax.fori_loop(0, nb, block, (zero16, zero16))

    a00, a01 = process(l0, slt0, cnt_v[0])
    a10, a11 = process(l1, slt1, cnt_v[1])
    pbuf[pl.ds(0, 16)] = a00
    pbuf[pl.ds(16, 16)] = a01
    pbuf[pl.ds(32, 16)] = a10
    pbuf[pl.ds(48, 16)] = a11
    pltpu.sync_copy(pbuf, part.at[pl.ds(w * 64, 64)])

  @functools.partial(
      pl.kernel,
      out_type=(jax.ShapeDtypeStruct((16,), _f32),
                jax.ShapeDtypeStruct((2, 64), _f32),
                jax.ShapeDtypeStruct((2, 64), _f32)),
      mesh=_mesh,
      compiler_params=_cp,
      scratch_types=[
          pltpu.VMEM((NW * 64,), _f32),
          pltpu.VMEM((NW * 16,), _i32),
          pltpu.VMEM((32 * 16,), _f32),
          pltpu.VMEM((16,), _f32),
          pltpu.VMEM((16 * 64,), _f32),
          pltpu.VMEM((64,), _f32),
          pltpu.VMEM((64 * 64,), _f32),
          pltpu.VMEM((64,), _f32),
          pltpu.VMEM((64 * 64,), _f32),
          pltpu.VMEM((64,), _f32),
          pltpu.VMEM((32,), _f32),
          pltpu.VMEM((32 * 64,), _f32),
          pltpu.VMEM((64,), _f32),
          pltpu.VMEM((32,), _f32),
          pltpu.VMEM((16,), _f32),
          pltpu.VMEM((64,), _f32),
          pltpu.VMEM((64,), _f32),
          pltpu.VMEM((64,), _f32),
          pltpu.VMEM((64,), _f32),
          pltpu.VMEM((64,), _f32),
          pltpu.VMEM((2, 64), _f32),
          pltpu.VMEM((16,), _f32),
      ],
  )
  def _k_fin(part, cnts, W2, b2, Wp1, bp1, Wp2, bp2, Wp3, bp3, beta, Wq2, bq2,
             lout, z1o, z2o,
             pv, cv, W2v, b2v, Wp1v, bp1v, Wp2v, bp2v, Wp3v, bp3v,
             betav, Wq2v, bq2v, mbuf, hbuf, t1b, t2b, y0b, y1b, pb, zb, lb):
    cid = lax.axis_index("c")
    sid = lax.axis_index("s")

    @pl.when(jnp.logical_and(cid == 0, sid == 0))
    def _():
      pltpu.sync_copy(part, pv)
      pltpu.sync_copy(cnts, cv)
      pltpu.sync_copy(W2, W2v)
      pltpu.sync_copy(b2, b2v)
      pltpu.sync_copy(Wp1, Wp1v)
      pltpu.sync_copy(bp1, bp1v)
      pltpu.sync_copy(Wp2, Wp2v)
      pltpu.sync_copy(bp2, bp2v)
      pltpu.sync_copy(Wp3, Wp3v)
      pltpu.sync_copy(bp3, bp3v)
      pltpu.sync_copy(beta, betav)
      pltpu.sync_copy(Wq2, Wq2v)
      pltpu.sync_copy(bq2, bq2v)

      def matvec_to(xref, wref, nin, nout, bref, dst, relu):
        nacc = nout // 16

        def body(d, accs):
          xs = plsc.load_gather(xref, [_full16(d)])
          return tuple(accs[k] + xs * wref[pl.ds(d * nout + k * 16, 16)]
                       for k in range(nacc))

        accs = lax.fori_loop(
            0, nin, body,
            tuple(jnp.zeros((16,), _f32) for _ in range(nacc)))
        for k in range(nacc):
          v = accs[k] + bref[pl.ds(k * 16, 16)]
          if relu:
            v = jnp.maximum(v, 0.0)
          dst[pl.ds(k * 16, 16)] = v

      def accw(w, accs):
        return tuple(accs[k] + pv[pl.ds(w * 64 + k * 16, 16)]
                     for k in range(4))

      a = lax.fori_loop(0, NW, accw,
                        tuple(jnp.zeros((16,), _f32) for _ in range(4)))
      cacc = lax.fori_loop(0, NW,
                           lambda w, c: c + cv[pl.ds(w * 16, 16)],
                           jnp.zeros((16,), _i32))
      di0 = _rsqrt_deg(_full16(cacc[0]).astype(_f32))
      di1 = _rsqrt_deg(_full16(cacc[1]).astype(_f32))

      def project(lo, hi, di, ybuf):
        mbuf[pl.ds(0, 16)] = lo * di
        mbuf[pl.ds(16, 16)] = hi * di
        matvec_to(mbuf, W2v, 32, 16, b2v, hbuf, True)
        matvec_to(hbuf, Wp1v, 16, 64, bp1v, t1b, True)
        matvec_to(t1b, Wp2v, 64, 64, bp2v, t2b, True)
        matvec_to(t2b, Wp3v, 64, 64, bp3v, ybuf, False)

      project(a[0], a[1], di0, y0b)
      project(a[2], a[3], di1, y1b)

      betav[pl.ds(0, 16)] = jnp.maximum(betav[pl.ds(0, 16)], 0.0)
      betav[pl.ds(16, 16)] = jnp.maximum(betav[pl.ds(16, 16)], 0.0)
      matvec_to(betav, Wq2v, 32, 64, bq2v, pb, False)

      def dot64(aref, bref):
        s = jnp.zeros((16,), _f32)
        for k in range(4):
          s = s + aref[pl.ds(k * 16, 16)] * bref[pl.ds(k * 16, 16)]
        return _full16(jnp.sum(s), _f32)

      pn2 = dot64(pb, pb)

      def cos_with_p(yref):
        d = dot64(pb, yref)
        yn2 = dot64(yref, yref)
        prod = pn2 * yn2
        den = jnp.maximum(prod * _rsqrt_raw(prod), 1e-8)
        return d / den

      c0 = cos_with_p(y0b)
      c1 = cos_with_p(y1b)
      lb[pl.ds(0, 16)] = -0.5 * (c0 + c1)
      pltpu.sync_copy(lb, lout)

      for k in range(4):
        v = y0b[pl.ds(k * 16, 16)]
        zb[0, pl.ds(k * 16, 16)] = v
        zb[1, pl.ds(k * 16, 16)] = v
      pltpu.sync_copy(zb, z1o)
      for k in range(4):
        v = y1b[pl.ds(k * 16, 16)]
        zb[0, pl.ds(k * 16, 16)] = v
        zb[1, pl.ds(k * 16, 16)] = v
      pltpu.sync_copy(zb, z2o)

  return (_k_hist, _k_hred, _k_lists, _k_mslot, _k_m1, _k_m2, _k_fin)


def kernel(feats, edge_index, W1, b1, W2, b2, Wp1, bp1, Wp2, bp2, Wp3, bp3,
           Wq1, bq1, gamma, beta, Wq2, bq2):
  (_k_hist, _k_hred, _k_lists, _k_mslot, _k_m1, _k_m2, _k_fin) = _build()
  ei = edge_index.astype(jnp.int32)
  esrc = ei[0]
  edst = ei[1]
  feats = feats.astype(jnp.float32)
  hpart = _k_hist(esrc)
  dego = _k_hred(hpart)
  l0, l1, cnts = _k_lists(esrc, edst)
  msrc, mslt, mcnt, slt0, slt1, nsl = _k_mslot(esrc, edst, l0, l1, cnts)
  hfe = jnp.concatenate(
      [feats, jnp.zeros((N, FP - F - 1), jnp.float32), dego[:N, None]],
      axis=1).reshape(-1)
  m1out = _k_m1(msrc, mslt, mcnt, hfe, nsl)
  part = _k_m2(l0, l1, slt0, slt1, cnts, m1out, hfe,
               W1.reshape(-1), b1)
  lout, z1, z2 = _k_fin(part, cnts, W2.reshape(-1), b2, Wp1.reshape(-1), bp1,
                        Wp2.reshape(-1), bp2, Wp3.reshape(-1), bp3,
                        beta, Wq2.reshape(-1), bq2)
  return lout[0], z1, z2
